# Initial kernel scaffold; baseline (speedup 1.0000x reference)
import functools

import jax
import jax.numpy as jnp
from jax import lax
from jax.experimental import pallas as pl
from jax.experimental.pallas import tpu as pltpu
from jax.experimental.pallas import tpu_sc as plsc

N = 50000
E = 800000
H = 64
HH = 32
CH = 128
NSUB = 16
EPT = 50048
E_PAD = EPT * NSUB
NPT = N // NSUB
BN = 2000
BE = 8000


def _iota16():
    return lax.iota(jnp.int32, 16)


def _enc0_body(x_ref, w_ref, as_ref, ad_ref, hpa_ref, hpb_ref, sv_ref, dv_ref):
    hp = jnp.dot(x_ref[...], w_ref[...], preferred_element_type=jnp.float32)
    hpa_ref[...] = hp[:, :HH]
    hpb_ref[...] = hp[:, HH:]
    sv_ref[...] = jnp.dot(hp, as_ref[...], preferred_element_type=jnp.float32)
    dv_ref[...] = jnp.dot(hp, ad_ref[...], preferred_element_type=jnp.float32)


def _enc_body(accA_ref, accB_ref, den_ref, b_ref, w_ref, as_ref, ad_ref,
              hpa_ref, hpb_ref, sv_ref, dv_ref):
    d = den_ref[:, 0:1]
    b = b_ref[...]
    ha = jax.nn.relu(accA_ref[...] / d + b[:, :HH])
    hb = jax.nn.relu(accB_ref[...] / d + b[:, HH:])
    w = w_ref[...]
    hp = (jnp.dot(ha, w[:HH, :], preferred_element_type=jnp.float32)
          + jnp.dot(hb, w[HH:, :], preferred_element_type=jnp.float32))
    hpa_ref[...] = hp[:, :HH]
    hpb_ref[...] = hp[:, HH:]
    sv_ref[...] = jnp.dot(hp, as_ref[...], preferred_element_type=jnp.float32)
    dv_ref[...] = jnp.dot(hp, ad_ref[...], preferred_element_type=jnp.float32)


def _sigmoid(x):
    return 1.0 / (1.0 + jnp.exp(-x))


def _heads_body(accA_ref, accB_ref, den_ref, b_ref, w1a_ref, w1b_ref, b1_ref,
                vw1_ref, vb1_ref, vw2_ref, vb2_ref,
                lw1_ref, lb1_ref, lw2_ref, lb2_ref,
                nf_ref, g1_ref, g2_ref, via_ref, lay_ref):
    d = den_ref[:, 0:1]
    b = b_ref[...]
    nfa = accA_ref[...] / d + b[:, :HH]
    nfb = accB_ref[...] / d + b[:, HH:]
    nf = jnp.concatenate([nfa, nfb], axis=-1)
    nf_ref[...] = nf
    g1_ref[...] = jnp.dot(nf, w1a_ref[...], preferred_element_type=jnp.float32) + b1_ref[...]
    g2_ref[...] = jnp.dot(nf, w1b_ref[...], preferred_element_type=jnp.float32)
    v = jax.nn.relu(jnp.dot(nf, vw1_ref[...], preferred_element_type=jnp.float32) + vb1_ref[...])
    via_ref[...] = _sigmoid(jnp.dot(v, vw2_ref[...], preferred_element_type=jnp.float32) + vb2_ref[...])
    l = jax.nn.relu(jnp.dot(nf, lw1_ref[...], preferred_element_type=jnp.float32) + lb1_ref[...])
    lg = jnp.dot(l, lw2_ref[...], preferred_element_type=jnp.float32) + lb2_ref[...]
    m = jnp.max(lg, axis=-1, keepdims=True)
    p = jnp.exp(lg - m)
    lay_ref[...] = p / jnp.sum(p, axis=-1, keepdims=True)


def _path_body(zs_ref, zd_ref, w2_ref, b2_ref, w3_ref, b3_ref, out_ref):
    p = jax.nn.relu(zs_ref[...] + zd_ref[...])
    p2 = jax.nn.relu(jnp.dot(p, w2_ref[...], preferred_element_type=jnp.float32) + b2_ref[...])
    out_ref[...] = _sigmoid(jnp.dot(p2, w3_ref[...], preferred_element_type=jnp.float32) + b3_ref[...])


def _full(shape):
    return pl.BlockSpec(shape, lambda i: tuple(0 for _ in shape))


def _enc0_call(x, w, a_s, a_d):
    return pl.pallas_call(
        _enc0_body,
        grid=(N // BN,),
        in_specs=[pl.BlockSpec((BN, 4), lambda i: (i, 0)),
                  _full((4, H)), _full((H, 1)), _full((H, 1))],
        out_specs=[pl.BlockSpec((BN, HH), lambda i: (i, 0)),
                   pl.BlockSpec((BN, HH), lambda i: (i, 0)),
                   pl.BlockSpec((BN, 1), lambda i: (i, 0)),
                   pl.BlockSpec((BN, 1), lambda i: (i, 0))],
        out_shape=[jax.ShapeDtypeStruct((N, HH), jnp.float32),
                   jax.ShapeDtypeStruct((N, HH), jnp.float32),
                   jax.ShapeDtypeStruct((N, 1), jnp.float32),
                   jax.ShapeDtypeStruct((N, 1), jnp.float32)],
    )(x, w, a_s, a_d)


def _enc_call(accA, accB, den, b, w, a_s, a_d):
    return pl.pallas_call(
        _enc_body,
        grid=(N // BN,),
        in_specs=[pl.BlockSpec((BN, HH), lambda i: (i, 0)),
                  pl.BlockSpec((BN, HH), lambda i: (i, 0)),
                  pl.BlockSpec((BN, 8), lambda i: (i, 0)),
                  _full((1, H)), _full((H, H)), _full((H, 1)), _full((H, 1))],
        out_specs=[pl.BlockSpec((BN, HH), lambda i: (i, 0)),
                   pl.BlockSpec((BN, HH), lambda i: (i, 0)),
                   pl.BlockSpec((BN, 1), lambda i: (i, 0)),
                   pl.BlockSpec((BN, 1), lambda i: (i, 0))],
        out_shape=[jax.ShapeDtypeStruct((N, HH), jnp.float32),
                   jax.ShapeDtypeStruct((N, HH), jnp.float32),
                   jax.ShapeDtypeStruct((N, 1), jnp.float32),
                   jax.ShapeDtypeStruct((N, 1), jnp.float32)],
    )(accA, accB, den, b, w, a_s, a_d)


def _heads_call(accA, accB, den, b, w1a, w1b, b1, vw1, vb1, vw2, vb2,
                lw1, lb1, lw2, lb2):
    return pl.pallas_call(
        _heads_body,
        grid=(N // BN,),
        in_specs=[pl.BlockSpec((BN, HH), lambda i: (i, 0)),
                  pl.BlockSpec((BN, HH), lambda i: (i, 0)),
                  pl.BlockSpec((BN, 8), lambda i: (i, 0)),
                  _full((1, H)), _full((H, H)), _full((H, H)), _full((1, H)),
                  _full((H, HH)), _full((1, HH)), _full((HH, 1)), _full((1, 1)),
                  _full((H, HH)), _full((1, HH)), _full((HH, 4)), _full((1, 4))],
        out_specs=[pl.BlockSpec((BN, H), lambda i: (i, 0)),
                   pl.BlockSpec((BN, H), lambda i: (i, 0)),
                   pl.BlockSpec((BN, H), lambda i: (i, 0)),
                   pl.BlockSpec((BN, 1), lambda i: (i, 0)),
                   pl.BlockSpec((BN, 4), lambda i: (i, 0))],
        out_shape=[jax.ShapeDtypeStruct((N, H), jnp.float32),
                   jax.ShapeDtypeStruct((N, H), jnp.float32),
                   jax.ShapeDtypeStruct((N, H), jnp.float32),
                   jax.ShapeDtypeStruct((N, 1), jnp.float32),
                   jax.ShapeDtypeStruct((N, 4), jnp.float32)],
    )(accA, accB, den, b, w1a, w1b, b1, vw1, vb1, vw2, vb2, lw1, lb1, lw2, lb2)


def _path_call(zs, zd, w2, b2, w3, b3):
    return pl.pallas_call(
        _path_body,
        grid=(E // BE,),
        in_specs=[pl.BlockSpec((BE, H), lambda i: (i, 0)),
                  pl.BlockSpec((BE, H), lambda i: (i, 0)),
                  _full((H, HH)), _full((1, HH)), _full((HH, 1)), _full((1, 1))],
        out_specs=pl.BlockSpec((BE, 1), lambda i: (i, 0)),
        out_shape=jax.ShapeDtypeStruct((E, 1), jnp.float32),
    )(zs, zd, w2, b2, w3, b3)


_MESH = plsc.VectorSubcoreMesh(core_axis_name="c", subcore_axis_name="s")


def _gat_edge_groups(as_t, ad_t, sbuf, dbuf, wbuf, w8, shift, valid_from, do_den):
    for g in range(CH // 16):
        row0 = g * 16
        sv = sbuf[pl.ds(row0, 16)]
        dv = dbuf[pl.ds(row0, 16)]
        a = plsc.load_gather(as_t, [sv]) + plsc.load_gather(ad_t, [dv])
        e = jnp.where(a > 0, a, 0.2 * a)
        w = jnp.exp(e - shift)
        valid = (row0 + _iota16()) < valid_from
        w = jnp.where(valid, w, 0.0)
        wbuf[pl.ds(row0, 16)] = w
        if do_den:
            f = row0 + _iota16()
            plsc.store_scatter(w8, [f, jnp.zeros((16,), jnp.int32)], w)


def _gat_scale_scatter(hp_ref, acc_sh, den_sh, sbuf, dbuf, wbuf, w8, rows,
                       sem, do_den):
    pltpu.async_copy(hp_ref.at[sbuf], rows, sem).wait()
    for g in range(CH // 16):
        wv = wbuf[pl.ds(g * 16, 16)]
        for r in range(16):
            row = g * 16 + r
            wb = jnp.full((16,), wv[r], jnp.float32)
            rows[row, 0:16] = rows[row, 0:16] * wb
            rows[row, 16:32] = rows[row, 16:32] * wb
    pltpu.sync_copy(rows, acc_sh.at[dbuf], add=True)
    if do_den:
        pltpu.sync_copy(w8, den_sh.at[dbuf], add=True)


def _gat_body(hpa, hpb, asv, adv, src, dst, acc_out, den_out,
              as_t, ad_t, sbuf, dbuf, wbuf, w8, rows, acc_sh, den_sh, sem):
    c = lax.axis_index("c")
    t = lax.axis_index("s")

    pltpu.sync_copy(asv, as_t)
    pltpu.sync_copy(adv, ad_t)

    def _mxstep(i, carry):
        ma, mb = carry
        return (jnp.maximum(ma, as_t[pl.ds(i * 16, 16)]),
                jnp.maximum(mb, ad_t[pl.ds(i * 16, 16)]))

    neg = jnp.full((16,), -1e30, jnp.float32)
    ma, mb = lax.fori_loop(0, N // 16, _mxstep, (neg, neg))
    shift_s = jnp.maximum(jnp.max(ma) + jnp.max(mb), 0.0)
    shift = jnp.full((16,), shift_s, jnp.float32)

    zf = jnp.zeros((16,), jnp.float32)
    for rr in range(CH):
        rows[rr, 0:16] = zf
        rows[rr, 16:32] = zf
    for i in range(CH * 8 // 16):
        f = 16 * i + _iota16()
        plsc.store_scatter(w8, [lax.shift_right_logical(f, 3),
                                jnp.bitwise_and(f, 7)], zf)

    r0 = t * NPT
    for j in range(NPT // 125):
        pltpu.sync_copy(rows.at[pl.ds(0, 125), :],
                        acc_sh.at[pl.ds(r0 + j * 125, 125), :])

    @pl.when(c == 0)
    def _zero_den():
        for j in range(NPT // 125):
            pltpu.sync_copy(w8.at[pl.ds(0, 125), :],
                            den_sh.at[pl.ds(r0 + j * 125, 125), :])

    plsc.subcore_barrier()

    big = jnp.int32(1 << 30)

    def _half(hp_ref, out_idx, do_den):
        ebase = t * EPT

        def _echunk(k, _):
            off = ebase + k * CH
            pltpu.sync_copy(src.at[pl.ds(off, CH)], sbuf)
            pltpu.sync_copy(dst.at[pl.ds(off, CH)], dbuf)
            vfrom = jnp.minimum(jnp.maximum(E - off, 0), big)
            _gat_edge_groups(as_t, ad_t, sbuf, dbuf, wbuf, w8, shift,
                             vfrom, do_den)
            _gat_scale_scatter(hp_ref, acc_sh, den_sh, sbuf, dbuf, wbuf, w8,
                               rows, sem, do_den)
            return 0

        lax.fori_loop(0, EPT // CH, _echunk, 0)

        nbase = t * NPT

        def _schunk(k, _):
            j0 = k * CH
            for g in range(CH // 16):
                row0 = g * 16
                loc = j0 + row0 + _iota16()
                ids = jnp.where(loc < NPT, nbase + loc, 0)
                sbuf[pl.ds(row0, 16)] = ids
                dbuf[pl.ds(row0, 16)] = ids
            vfrom = NPT - j0
            _gat_edge_groups(as_t, ad_t, sbuf, dbuf, wbuf, w8, shift,
                             vfrom, do_den)
            _gat_scale_scatter(hp_ref, acc_sh, den_sh, sbuf, dbuf, wbuf, w8,
                               rows, sem, do_den)
            return 0

        lax.fori_loop(0, (NPT + CH - 1) // CH, _schunk, 0)

        plsc.subcore_barrier()
        pltpu.sync_copy(acc_sh.at[pl.ds(r0, NPT), :],
                        acc_out.at[out_idx, pl.ds(r0, NPT), :])
        if do_den:
            pltpu.sync_copy(den_sh.at[pl.ds(r0, NPT), :],
                            den_out.at[pl.ds(r0, NPT), :])

    @pl.when(c == 0)
    def _c0():
        _half(hpa, 0, True)

    @pl.when(c == 1)
    def _c1():
        _half(hpb, 1, False)


@functools.partial(
    pl.kernel,
    out_type=[jax.ShapeDtypeStruct((2, N, HH), jnp.float32),
              jax.ShapeDtypeStruct((N, 8), jnp.float32)],
    mesh=_MESH,
    scratch_types=[
        pltpu.VMEM((N,), jnp.float32),
        pltpu.VMEM((N,), jnp.float32),
        pltpu.VMEM((CH,), jnp.int32),
        pltpu.VMEM((CH,), jnp.int32),
        pltpu.VMEM((CH,), jnp.float32),
        pltpu.VMEM((CH, 8), jnp.float32),
        pltpu.VMEM((CH, HH), jnp.float32),
        pltpu.VMEM_SHARED((N, HH), jnp.float32),
        pltpu.VMEM_SHARED((N, 8), jnp.float32),
        pltpu.SemaphoreType.DMA,
    ],
)
def _gat_kernel(hpa, hpb, asv, adv, src, dst, acc_out, den_out,
                as_t, ad_t, sbuf, dbuf, wbuf, w8, rows, acc_sh, den_sh, sem):
    _gat_body(hpa, hpb, asv, adv, src, dst, acc_out, den_out,
              as_t, ad_t, sbuf, dbuf, wbuf, w8, rows, acc_sh, den_sh, sem)


EPT2 = E // 32


def _pair_body(g1, g2, src, dst, zs, zd, sbuf, dbuf, rows1, rows2, sem):
    c = lax.axis_index("c")
    t = lax.axis_index("s")
    wid = t * 2 + c
    base = wid * EPT2

    def _chunk(off, nreal):
        if nreal < CH:
            zi = jnp.zeros((16,), jnp.int32)
            for g in range(CH // 16):
                sbuf[pl.ds(g * 16, 16)] = zi
                dbuf[pl.ds(g * 16, 16)] = zi
        pltpu.sync_copy(src.at[pl.ds(off, nreal)], sbuf.at[pl.ds(0, nreal)])
        pltpu.sync_copy(dst.at[pl.ds(off, nreal)], dbuf.at[pl.ds(0, nreal)])
        pltpu.async_copy(g1.at[sbuf], rows1, sem).wait()
        pltpu.async_copy(g2.at[dbuf], rows2, sem).wait()
        pltpu.sync_copy(rows1.at[pl.ds(0, nreal), :], zs.at[pl.ds(off, nreal), :])
        pltpu.sync_copy(rows2.at[pl.ds(0, nreal), :], zd.at[pl.ds(off, nreal), :])

    def _step(k, _):
        _chunk(base + k * CH, CH)
        return 0

    nfull = EPT2 // CH
    lax.fori_loop(0, nfull, _step, 0)
    _chunk(base + nfull * CH, EPT2 - nfull * CH)


@functools.partial(
    pl.kernel,
    out_type=[jax.ShapeDtypeStruct((E, H), jnp.float32),
              jax.ShapeDtypeStruct((E, H), jnp.float32)],
    mesh=_MESH,
    scratch_types=[
        pltpu.VMEM((CH,), jnp.int32),
        pltpu.VMEM((CH,), jnp.int32),
        pltpu.VMEM((CH, H), jnp.float32),
        pltpu.VMEM((CH, H), jnp.float32),
        pltpu.SemaphoreType.DMA,
    ],
)
def _pair_kernel(g1, g2, src, dst, zs, zd, sbuf, dbuf, rows1, rows2, sem):
    _pair_body(g1, g2, src, dst, zs, zd, sbuf, dbuf, rows1, rows2, sem)


def kernel(x, edge_index, W0, a_s0, a_d0, b0, Ws, a_ss, a_ds, bs,
           pp_W1, pp_b1, pp_W2, pp_b2, pp_W3, pp_b3,
           vp_W1, vp_b1, vp_W2, vp_b2, lp_W1, lp_b1, lp_W2, lp_b2):
    src = edge_index[0]
    dst = edge_index[1]
    pad = jnp.zeros((E_PAD - E,), jnp.int32)
    src_p = jnp.concatenate([src, pad])
    dst_p = jnp.concatenate([dst, pad])

    hpa, hpb, sv, dv = _enc0_call(x, W0, a_s0.reshape(H, 1), a_d0.reshape(H, 1))
    acc2, den = _gat_kernel(hpa, hpb, sv.reshape(N), dv.reshape(N), src_p, dst_p)

    biases = [b0, bs[0], bs[1]]
    for i in range(3):
        hpa, hpb, sv, dv = _enc_call(
            acc2[0], acc2[1], den, biases[i].reshape(1, H), Ws[i],
            a_ss[i].reshape(H, 1), a_ds[i].reshape(H, 1))
        acc2, den = _gat_kernel(hpa, hpb, sv.reshape(N), dv.reshape(N),
                                src_p, dst_p)

    nf, g1, g2, vias, layers = _heads_call(
        acc2[0], acc2[1], den, bs[2].reshape(1, H),
        pp_W1[:H, :], pp_W1[H:, :], pp_b1.reshape(1, H),
        vp_W1, vp_b1.reshape(1, HH), vp_W2, vp_b2.reshape(1, 1),
        lp_W1, lp_b1.reshape(1, HH), lp_W2, lp_b2.reshape(1, 4))

    zs, zd = _pair_kernel(g1, g2, src, dst)
    paths = _path_call(zs, zd, pp_W2, pp_b2.reshape(1, HH),
                       pp_W3, pp_b3.reshape(1, 1))

    return (paths.reshape(E), vias.reshape(N), layers, nf)


# speedup vs baseline: 9.9674x; 9.9674x over previous
import functools

import jax
import jax.numpy as jnp
from jax import lax
from jax.experimental import pallas as pl
from jax.experimental.pallas import tpu as pltpu
from jax.experimental.pallas import tpu_sc as plsc

N = 50000
NP = 50048
E = 800000
H = 64
HH = 16
DW = 32
CH = 128
NSUB = 16
E_PAD = 802816
EPW = E_PAD // 32
SELF_STRIDE = 1664
SELF_LEN = 32 * SELF_STRIDE
W_LEN = E_PAD + SELF_LEN
EPG = E_PAD // NSUB
SPG = SELF_LEN // NSUB
NPT = NP // NSUB
ND = 1664
NDT = ND // NSUB
BN = 3128
BE = 8000


def _iota16():
    return lax.iota(jnp.int32, 16)


def _enc0_body(x_ref, w_ref, as_ref, ad_ref, hp4_ref, sv_ref, dv_ref):
    hp = jnp.dot(x_ref[...], w_ref[...], preferred_element_type=jnp.float32)
    for q in range(4):
        hp4_ref[q] = hp[:, q * HH:(q + 1) * HH]
    sv_ref[...] = jnp.dot(hp, as_ref[...], preferred_element_type=jnp.float32)
    dv_ref[...] = jnp.dot(hp, ad_ref[...], preferred_element_type=jnp.float32)


def _enc_body(q0_ref, q1_ref, q2_ref, q3_ref, dena_ref, denb_ref, b_ref,
              w_ref, as_ref, ad_ref, hp4_ref, sv_ref, dv_ref):
    d = dena_ref[...] + denb_ref[...]
    b = b_ref[...]
    w = w_ref[...]
    hp = None
    for q, qref in enumerate((q0_ref, q1_ref, q2_ref, q3_ref)):
        hq = jax.nn.relu(qref[...] / d + b[:, q * HH:(q + 1) * HH])
        part = jnp.dot(hq, w[q * HH:(q + 1) * HH, :],
                       preferred_element_type=jnp.float32)
        hp = part if hp is None else hp + part
    for q in range(4):
        hp4_ref[q] = hp[:, q * HH:(q + 1) * HH]
    sv_ref[...] = jnp.dot(hp, as_ref[...], preferred_element_type=jnp.float32)
    dv_ref[...] = jnp.dot(hp, ad_ref[...], preferred_element_type=jnp.float32)


def _sigmoid(x):
    return 1.0 / (1.0 + jnp.exp(-x))


def _heads_body(q0_ref, q1_ref, q2_ref, q3_ref, dena_ref, denb_ref, b_ref,
                w1a_ref, w1b_ref, b1_ref,
                vw1_ref, vb1_ref, vw2_ref, vb2_ref,
                lw1_ref, lb1_ref, lw2_ref, lb2_ref,
                nf_ref, g1_ref, g2_ref, via_ref, lay_ref):
    d = dena_ref[...] + denb_ref[...]
    b = b_ref[...]
    nf = jnp.concatenate(
        [qref[...] / d + b[:, q * HH:(q + 1) * HH]
         for q, qref in enumerate((q0_ref, q1_ref, q2_ref, q3_ref))], axis=-1)
    nf_ref[...] = nf
    g1_ref[...] = jnp.dot(nf, w1a_ref[...], preferred_element_type=jnp.float32) + b1_ref[...]
    g2_ref[...] = jnp.dot(nf, w1b_ref[...], preferred_element_type=jnp.float32)
    v = jax.nn.relu(jnp.dot(nf, vw1_ref[...], preferred_element_type=jnp.float32) + vb1_ref[...])
    via_ref[...] = _sigmoid(jnp.dot(v, vw2_ref[...], preferred_element_type=jnp.float32) + vb2_ref[...])
    l = jax.nn.relu(jnp.dot(nf, lw1_ref[...], preferred_element_type=jnp.float32) + lb1_ref[...])
    lg = jnp.dot(l, lw2_ref[...], preferred_element_type=jnp.float32) + lb2_ref[...]
    m = jnp.max(lg, axis=-1, keepdims=True)
    p = jnp.exp(lg - m)
    lay_ref[...] = p / jnp.sum(p, axis=-1, keepdims=True)


def _path_body(zs_ref, zd_ref, w2_ref, b2_ref, w3_ref, b3_ref, out_ref):
    p = jax.nn.relu(zs_ref[...] + zd_ref[...])
    p2 = jax.nn.relu(jnp.dot(p, w2_ref[...], preferred_element_type=jnp.float32) + b2_ref[...])
    out_ref[...] = _sigmoid(jnp.dot(p2, w3_ref[...], preferred_element_type=jnp.float32) + b3_ref[...])


def _full(shape):
    return pl.BlockSpec(shape, lambda i: tuple(0 for _ in shape))


def _bn(width):
    return pl.BlockSpec((BN, width), lambda i: (i, 0))


def _enc0_call(x, w, a_s, a_d):
    return pl.pallas_call(
        _enc0_body,
        grid=(NP // BN,),
        in_specs=[_bn(4), _full((4, H)), _full((H, 1)), _full((H, 1))],
        out_specs=[pl.BlockSpec((4, BN, HH), lambda i: (0, i, 0)),
                   _bn(1), _bn(1)],
        out_shape=[jax.ShapeDtypeStruct((4, NP, HH), jnp.float32),
                   jax.ShapeDtypeStruct((NP, 1), jnp.float32),
                   jax.ShapeDtypeStruct((NP, 1), jnp.float32)],
    )(x, w, a_s, a_d)


def _enc_call(q0, q1, q2, q3, den_a, den_b, b, w, a_s, a_d):
    return pl.pallas_call(
        _enc_body,
        grid=(NP // BN,),
        in_specs=[_bn(HH), _bn(HH), _bn(HH), _bn(HH), _bn(1), _bn(1),
                  _full((1, H)), _full((H, H)), _full((H, 1)), _full((H, 1))],
        out_specs=[pl.BlockSpec((4, BN, HH), lambda i: (0, i, 0)),
                   _bn(1), _bn(1)],
        out_shape=[jax.ShapeDtypeStruct((4, NP, HH), jnp.float32),
                   jax.ShapeDtypeStruct((NP, 1), jnp.float32),
                   jax.ShapeDtypeStruct((NP, 1), jnp.float32)],
    )(q0, q1, q2, q3, den_a, den_b, b, w, a_s, a_d)


def _heads_call(q0, q1, q2, q3, den_a, den_b, b, w1a, w1b, b1,
                vw1, vb1, vw2, vb2, lw1, lb1, lw2, lb2):
    return pl.pallas_call(
        _heads_body,
        grid=(NP // BN,),
        in_specs=[_bn(HH), _bn(HH), _bn(HH), _bn(HH), _bn(1), _bn(1),
                  _full((1, H)), _full((H, H)), _full((H, H)), _full((1, H)),
                  _full((H, 32)), _full((1, 32)), _full((32, 1)), _full((1, 1)),
                  _full((H, 32)), _full((1, 32)), _full((32, 4)), _full((1, 4))],
        out_specs=[_bn(H), _bn(H), _bn(H), _bn(1), _bn(4)],
        out_shape=[jax.ShapeDtypeStruct((NP, H), jnp.float32),
                   jax.ShapeDtypeStruct((NP, H), jnp.float32),
                   jax.ShapeDtypeStruct((NP, H), jnp.float32),
                   jax.ShapeDtypeStruct((NP, 1), jnp.float32),
                   jax.ShapeDtypeStruct((NP, 4), jnp.float32)],
    )(q0, q1, q2, q3, den_a, den_b, b, w1a, w1b, b1,
      vw1, vb1, vw2, vb2, lw1, lb1, lw2, lb2)


def _path_call(zs, zd, w2, b2, w3, b3):
    return pl.pallas_call(
        _path_body,
        grid=(E // BE,),
        in_specs=[pl.BlockSpec((BE, H), lambda i: (i, 0)),
                  pl.BlockSpec((BE, H), lambda i: (i, 0)),
                  _full((H, 32)), _full((1, 32)), _full((32, 1)), _full((1, 1))],
        out_specs=pl.BlockSpec((BE, 1), lambda i: (i, 0)),
        out_shape=jax.ShapeDtypeStruct((E, 1), jnp.float32),
    )(zs, zd, w2, b2, w3, b3)


_MESH = plsc.VectorSubcoreMesh(core_axis_name="c", subcore_axis_name="s")
_SC_PARAMS = pltpu.CompilerParams(needs_layout_passes=False,
                                  use_tc_tiling_on_sc=False)


def _w_groups(as_t, ad_t, sbuf, dbuf, wbuf, w32, cbuf, d2buf, shift,
              valid_from):
    zf16 = jnp.zeros((16,), jnp.float32)
    for g in range(CH // 16):
        row0 = g * 16
        sv = sbuf[pl.ds(row0, 16)]
        dv = dbuf[pl.ds(row0, 16)]
        a = plsc.load_gather(as_t, [sv]) + plsc.load_gather(ad_t, [dv])
        e = jnp.where(a > 0, a, 0.2 * a)
        w = jnp.exp(e - shift)
        valid = (row0 + _iota16()) < valid_from
        w = jnp.where(valid, w, 0.0)
        wbuf[pl.ds(row0, 16)] = w
        f = row0 + _iota16()
        oldcol = cbuf[pl.ds(row0, 16)]
        plsc.store_scatter(w32, [f, oldcol], zf16)
        col = jnp.bitwise_and(dv, DW - 1)
        plsc.store_scatter(w32, [f, col], w)
        cbuf[pl.ds(row0, 16)] = col
        d2buf[pl.ds(row0, 16)] = lax.shift_right_logical(dv, 5)


def _wden_body(asv, adv, src, dst, w_all, denf,
               as_t, ad_t, sbuf, dbuf, wbuf, w32, cbuf, d2buf, den_sh):
    c = lax.axis_index("c")
    t = lax.axis_index("s")
    wid = t * 2 + c

    pltpu.sync_copy(asv.at[pl.ds(0, N)], as_t)
    pltpu.sync_copy(adv.at[pl.ds(0, N)], ad_t)

    def _mxstep(i, carry):
        ma, mb = carry
        return (jnp.maximum(ma, as_t[pl.ds(i * 16, 16)]),
                jnp.maximum(mb, ad_t[pl.ds(i * 16, 16)]))

    neg = jnp.full((16,), -1e30, jnp.float32)
    ma, mb = lax.fori_loop(0, N // 16, _mxstep, (neg, neg))

    wbuf[pl.ds(0, 16)] = ma
    wbuf[pl.ds(16, 16)] = mb
    m0 = plsc.load_gather(wbuf, [jnp.zeros((16,), jnp.int32)])
    m1 = plsc.load_gather(wbuf, [jnp.full((16,), 16, jnp.int32)])
    for i in range(1, 16):
        m0 = jnp.maximum(m0, plsc.load_gather(
            wbuf, [jnp.full((16,), i, jnp.int32)]))
        m1 = jnp.maximum(m1, plsc.load_gather(
            wbuf, [jnp.full((16,), 16 + i, jnp.int32)]))
    shift = jnp.maximum(m0 + m1, jnp.zeros((16,), jnp.float32))

    zf = jnp.zeros((16,), jnp.float32)
    zi = jnp.zeros((16,), jnp.int32)
    for rr in range(CH):
        w32[rr, 0:16] = zf
        w32[rr, 16:32] = zf
    for g in range(CH // 16):
        cbuf[pl.ds(g * 16, 16)] = zi
    pltpu.sync_copy(w32.at[pl.ds(0, NDT), :],
                    den_sh.at[pl.ds(t * NDT, NDT), :])

    plsc.subcore_barrier()

    big = jnp.int32(1 << 30)
    ebase = wid * EPW

    def _echunk(k, _):
        off = ebase + k * CH
        pltpu.sync_copy(src.at[pl.ds(off, CH)], sbuf)
        pltpu.sync_copy(dst.at[pl.ds(off, CH)], dbuf)
        vfrom = jnp.minimum(jnp.maximum(E - off, 0), big)
        _w_groups(as_t, ad_t, sbuf, dbuf, wbuf, w32, cbuf, d2buf, shift,
                  vfrom)
        pltpu.sync_copy(wbuf, w_all.at[pl.ds(off, CH)])
        pltpu.sync_copy(w32, den_sh.at[d2buf], add=True)
        return 0

    lax.fori_loop(0, EPW // CH, _echunk, 0)

    sbase = wid * SELF_STRIDE

    def _schunk(k, _):
        j0 = k * CH
        for g in range(CH // 16):
            row0 = g * 16
            ids = sbase + j0 + row0 + _iota16()
            ids = jnp.where(ids < N, ids, 0)
            sbuf[pl.ds(row0, 16)] = ids
            dbuf[pl.ds(row0, 16)] = ids
        vfrom = jnp.minimum(SELF_STRIDE, N - sbase) - j0
        _w_groups(as_t, ad_t, sbuf, dbuf, wbuf, w32, cbuf, d2buf, shift,
                  vfrom)
        pltpu.sync_copy(wbuf, w_all.at[pl.ds(E_PAD + sbase + j0, CH)])
        pltpu.sync_copy(w32, den_sh.at[d2buf], add=True)
        return 0

    lax.fori_loop(0, SELF_STRIDE // CH, _schunk, 0)

    plsc.subcore_barrier()
    pltpu.sync_copy(den_sh.at[pl.ds(t * NDT, NDT), :],
                    denf.at[pl.ds(c * ND + t * NDT, NDT), :])


@functools.partial(
    pl.kernel,
    out_type=[jax.ShapeDtypeStruct((W_LEN,), jnp.float32),
              jax.ShapeDtypeStruct((2 * ND, DW), jnp.float32)],
    mesh=_MESH,
    compiler_params=_SC_PARAMS,
    scratch_types=[
        pltpu.VMEM((N,), jnp.float32),
        pltpu.VMEM((N,), jnp.float32),
        pltpu.VMEM((CH,), jnp.int32),
        pltpu.VMEM((CH,), jnp.int32),
        pltpu.VMEM((CH,), jnp.float32),
        pltpu.VMEM((CH, DW), jnp.float32),
        pltpu.VMEM((CH,), jnp.int32),
        pltpu.VMEM((CH,), jnp.int32),
        pltpu.VMEM_SHARED((ND, DW), jnp.float32),
    ],
)
def _wden_kernel(asv, adv, src, dst, w_all, denf,
                 as_t, ad_t, sbuf, dbuf, wbuf, w32, cbuf, d2buf, den_sh):
    _wden_body(asv, adv, src, dst, w_all, denf,
               as_t, ad_t, sbuf, dbuf, wbuf, w32, cbuf, d2buf, den_sh)


def _acc_body(hp4f, src, dst, w_all, accf,
              sbuf, dbuf, wbuf, rows, acc_sh, sem):
    c = lax.axis_index("c")
    t = lax.axis_index("s")
    cN = c * NP
    r0 = t * NPT
    zf = jnp.zeros((16,), jnp.float32)
    ebase = t * EPG
    sbase = t * SPG

    def _chunk_tail(pofs):
        for g in range(CH // 16):
            row0 = g * 16
            sbuf[pl.ds(row0, 16)] = sbuf[pl.ds(row0, 16)] + pofs
        pltpu.async_copy(hp4f.at[sbuf], rows, sem).wait()
        for g in range(CH // 16):
            wv = wbuf[pl.ds(g * 16, 16)]
            for r in range(16):
                row = g * 16 + r
                wb = jnp.full((16,), wv[r], jnp.float32)
                rows[row, 0:16] = rows[row, 0:16] * wb
        pltpu.sync_copy(rows, acc_sh.at[dbuf], add=True)

    for ps in (0, 1):
        pofs = 2 * ps * NP + cN

        for rr in range(CH):
            rows[rr, 0:16] = zf
        for j in range(NPT // CH):
            pltpu.sync_copy(rows, acc_sh.at[pl.ds(r0 + j * CH, CH), :])
        pltpu.sync_copy(rows.at[pl.ds(0, NPT % CH), :],
                        acc_sh.at[pl.ds(r0 + (NPT // CH) * CH, NPT % CH), :])

        plsc.subcore_barrier()

        def _echunk(k, _, pofs=pofs):
            off = ebase + k * CH
            pltpu.sync_copy(src.at[pl.ds(off, CH)], sbuf)
            pltpu.sync_copy(dst.at[pl.ds(off, CH)], dbuf)
            pltpu.sync_copy(w_all.at[pl.ds(off, CH)], wbuf)
            _chunk_tail(pofs)
            return 0

        lax.fori_loop(0, EPG // CH, _echunk, 0)

        def _schunk(k, _, pofs=pofs):
            j0 = k * CH
            for g in range(CH // 16):
                row0 = g * 16
                ids = sbase + j0 + row0 + _iota16()
                ids = jnp.where(ids < N, ids, 0)
                sbuf[pl.ds(row0, 16)] = ids
                dbuf[pl.ds(row0, 16)] = ids
            pltpu.sync_copy(w_all.at[pl.ds(E_PAD + sbase + j0, CH)], wbuf)
            _chunk_tail(pofs)
            return 0

        lax.fori_loop(0, SPG // CH, _schunk, 0)

        plsc.subcore_barrier()
        pltpu.sync_copy(acc_sh.at[pl.ds(r0, NPT), :],
                        accf.at[pl.ds(pofs + r0, NPT), :])


@functools.partial(
    pl.kernel,
    out_type=jax.ShapeDtypeStruct((4 * NP, HH), jnp.float32),
    mesh=_MESH,
    compiler_params=_SC_PARAMS,
    scratch_types=[
        pltpu.VMEM((CH,), jnp.int32),
        pltpu.VMEM((CH,), jnp.int32),
        pltpu.VMEM((CH,), jnp.float32),
        pltpu.VMEM((CH, HH), jnp.float32),
        pltpu.VMEM_SHARED((NP, HH), jnp.float32),
        pltpu.SemaphoreType.DMA,
    ],
)
def _acc_kernel(hp4f, src, dst, w_all, accf,
                sbuf, dbuf, wbuf, rows, acc_sh, sem):
    _acc_body(hp4f, src, dst, w_all, accf,
              sbuf, dbuf, wbuf, rows, acc_sh, sem)


EPT2 = E // 32


def _pair_body(g1, g2, src, dst, zs, zd, sbuf, dbuf, rows1, rows2, sem):
    c = lax.axis_index("c")
    t = lax.axis_index("s")
    wid = t * 2 + c
    base = wid * EPT2

    def _chunk(off, nreal):
        if nreal < CH:
            zi = jnp.zeros((16,), jnp.int32)
            for g in range(CH // 16):
                sbuf[pl.ds(g * 16, 16)] = zi
                dbuf[pl.ds(g * 16, 16)] = zi
        pltpu.sync_copy(src.at[pl.ds(off, nreal)], sbuf.at[pl.ds(0, nreal)])
        pltpu.sync_copy(dst.at[pl.ds(off, nreal)], dbuf.at[pl.ds(0, nreal)])
        pltpu.async_copy(g1.at[sbuf], rows1, sem).wait()
        pltpu.async_copy(g2.at[dbuf], rows2, sem).wait()
        pltpu.sync_copy(rows1.at[pl.ds(0, nreal), :], zs.at[pl.ds(off, nreal), :])
        pltpu.sync_copy(rows2.at[pl.ds(0, nreal), :], zd.at[pl.ds(off, nreal), :])

    def _step(k, _):
        _chunk(base + k * CH, CH)
        return 0

    nfull = EPT2 // CH
    lax.fori_loop(0, nfull, _step, 0)
    _chunk(base + nfull * CH, EPT2 - nfull * CH)


@functools.partial(
    pl.kernel,
    out_type=[jax.ShapeDtypeStruct((E, H), jnp.float32),
              jax.ShapeDtypeStruct((E, H), jnp.float32)],
    mesh=_MESH,
    compiler_params=_SC_PARAMS,
    scratch_types=[
        pltpu.VMEM((CH,), jnp.int32),
        pltpu.VMEM((CH,), jnp.int32),
        pltpu.VMEM((CH, H), jnp.float32),
        pltpu.VMEM((CH, H), jnp.float32),
        pltpu.SemaphoreType.DMA,
    ],
)
def _pair_kernel(g1, g2, src, dst, zs, zd, sbuf, dbuf, rows1, rows2, sem):
    _pair_body(g1, g2, src, dst, zs, zd, sbuf, dbuf, rows1, rows2, sem)


def kernel(x, edge_index, W0, a_s0, a_d0, b0, Ws, a_ss, a_ds, bs,
           pp_W1, pp_b1, pp_W2, pp_b2, pp_W3, pp_b3,
           vp_W1, vp_b1, vp_W2, vp_b2, lp_W1, lp_b1, lp_W2, lp_b2):
    src = edge_index[0]
    dst = edge_index[1]
    pad = jnp.zeros((E_PAD - E,), jnp.int32)
    src_p = jnp.concatenate([src, pad])
    dst_p = jnp.concatenate([dst, pad])
    x_p = jnp.concatenate([x, jnp.zeros((NP - N, x.shape[1]), x.dtype)])

    hp4, sv, dv = _enc0_call(x_p, W0, a_s0.reshape(H, 1), a_d0.reshape(H, 1))

    w_xs = jnp.stack([Ws[0], Ws[1], Ws[2], Ws[2]])
    as_xs = jnp.concatenate([a_ss, a_ss[2:3]]).reshape(4, H, 1)
    ad_xs = jnp.concatenate([a_ds, a_ds[2:3]]).reshape(4, H, 1)
    b_xs = jnp.stack([b0, bs[0], bs[1], bs[2]]).reshape(4, 1, H)

    def _layer_step(carry, xs):
        hp4, sv, dv = carry[:3]
        w_all, denf = _wden_kernel(sv, dv, src_p, dst_p)
        accf = _acc_kernel(hp4.reshape(4 * NP, HH), src_p, dst_p, w_all)
        quarters = accf.reshape(4, NP, HH)
        den_a = denf[:ND].reshape(ND * DW, 1)[:NP]
        den_b = denf[ND:].reshape(ND * DW, 1)[:NP]
        hp4n, svn, dvn = _enc_call(
            quarters[0], quarters[1], quarters[2], quarters[3], den_a, den_b,
            xs["b"], xs["W"], xs["a_s"], xs["a_d"])
        return ((hp4n, svn.reshape(NP), dvn.reshape(NP), quarters,
                 den_a, den_b), None)

    init = (hp4, sv.reshape(NP), dv.reshape(NP),
            jnp.zeros((4, NP, HH), jnp.float32),
            jnp.zeros((NP, 1), jnp.float32),
            jnp.zeros((NP, 1), jnp.float32))
    carry, _ = lax.scan(
        _layer_step, init,
        {"W": w_xs, "a_s": as_xs, "a_d": ad_xs, "b": b_xs})
    quarters = carry[3]
    den_a, den_b = carry[4], carry[5]

    nf, g1, g2, vias, layers = _heads_call(
        quarters[0], quarters[1], quarters[2], quarters[3], den_a, den_b,
        bs[2].reshape(1, H),
        pp_W1[:H, :], pp_W1[H:, :], pp_b1.reshape(1, H),
        vp_W1, vp_b1.reshape(1, 32), vp_W2, vp_b2.reshape(1, 1),
        lp_W1, lp_b1.reshape(1, 32), lp_W2, lp_b2.reshape(1, 4))

    zs, zd = _pair_kernel(g1, g2, src, dst)
    paths = _path_call(zs, zd, pp_W2, pp_b2.reshape(1, 32),
                       pp_W3, pp_b3.reshape(1, 1))

    return (paths.reshape(E), vias.reshape(NP)[:N], layers[:N], nf[:N])


# speedup vs baseline: 16.0198x; 1.6072x over previous
import functools

import jax
import jax.numpy as jnp
from jax import lax
from jax.experimental import pallas as pl
from jax.experimental.pallas import tpu as pltpu
from jax.experimental.pallas import tpu_sc as plsc

N = 50000
NP = 50048
E = 800000
H = 64
HH = 16
DW = 32
CH = 128
NSUB = 16
E_PAD = 802816
EPW = E_PAD // 32
SELF_STRIDE = 1664
SELF_LEN = 32 * SELF_STRIDE
W_LEN = E_PAD + SELF_LEN
EPG = E_PAD // NSUB
SPG = SELF_LEN // NSUB
NPT = NP // NSUB
ND = 1664
NDT = ND // NSUB
BN = 3128
BE = 8000


def _iota16():
    return lax.iota(jnp.int32, 16)


def _enc0_body(x_ref, w_ref, as_ref, ad_ref, hp4_ref, sv_ref, dv_ref):
    hp = jnp.dot(x_ref[...], w_ref[...], preferred_element_type=jnp.float32)
    for q in range(4):
        hp4_ref[q] = hp[:, q * HH:(q + 1) * HH]
    sv_ref[...] = jnp.dot(hp, as_ref[...], preferred_element_type=jnp.float32)
    dv_ref[...] = jnp.dot(hp, ad_ref[...], preferred_element_type=jnp.float32)


def _enc_body(q0_ref, q1_ref, q2_ref, q3_ref, dena_ref, denb_ref, b_ref,
              w_ref, as_ref, ad_ref, hp4_ref, sv_ref, dv_ref):
    d = dena_ref[...] + denb_ref[...]
    b = b_ref[...]
    w = w_ref[...]
    hp = None
    for q, qref in enumerate((q0_ref, q1_ref, q2_ref, q3_ref)):
        hq = jax.nn.relu(qref[...] / d + b[:, q * HH:(q + 1) * HH])
        part = jnp.dot(hq, w[q * HH:(q + 1) * HH, :],
                       preferred_element_type=jnp.float32)
        hp = part if hp is None else hp + part
    for q in range(4):
        hp4_ref[q] = hp[:, q * HH:(q + 1) * HH]
    sv_ref[...] = jnp.dot(hp, as_ref[...], preferred_element_type=jnp.float32)
    dv_ref[...] = jnp.dot(hp, ad_ref[...], preferred_element_type=jnp.float32)


def _sigmoid(x):
    return 1.0 / (1.0 + jnp.exp(-x))


def _heads_body(q0_ref, q1_ref, q2_ref, q3_ref, dena_ref, denb_ref, b_ref,
                w1a_ref, w1b_ref, b1_ref,
                vw1_ref, vb1_ref, vw2_ref, vb2_ref,
                lw1_ref, lb1_ref, lw2_ref, lb2_ref,
                nf_ref, g1_ref, g2_ref, via_ref, lay_ref):
    d = dena_ref[...] + denb_ref[...]
    b = b_ref[...]
    nf = jnp.concatenate(
        [qref[...] / d + b[:, q * HH:(q + 1) * HH]
         for q, qref in enumerate((q0_ref, q1_ref, q2_ref, q3_ref))], axis=-1)
    nf_ref[...] = nf
    g1_ref[...] = jnp.dot(nf, w1a_ref[...], preferred_element_type=jnp.float32) + b1_ref[...]
    g2_ref[...] = jnp.dot(nf, w1b_ref[...], preferred_element_type=jnp.float32)
    v = jax.nn.relu(jnp.dot(nf, vw1_ref[...], preferred_element_type=jnp.float32) + vb1_ref[...])
    via_ref[...] = _sigmoid(jnp.dot(v, vw2_ref[...], preferred_element_type=jnp.float32) + vb2_ref[...])
    l = jax.nn.relu(jnp.dot(nf, lw1_ref[...], preferred_element_type=jnp.float32) + lb1_ref[...])
    lg = jnp.dot(l, lw2_ref[...], preferred_element_type=jnp.float32) + lb2_ref[...]
    m = jnp.max(lg, axis=-1, keepdims=True)
    p = jnp.exp(lg - m)
    lay_ref[...] = p / jnp.sum(p, axis=-1, keepdims=True)


def _path_body(zs_ref, zd_ref, w2_ref, b2_ref, w3_ref, b3_ref, out_ref):
    p = jax.nn.relu(zs_ref[...] + zd_ref[...])
    p2 = jax.nn.relu(jnp.dot(p, w2_ref[...], preferred_element_type=jnp.float32) + b2_ref[...])
    out_ref[...] = _sigmoid(jnp.dot(p2, w3_ref[...], preferred_element_type=jnp.float32) + b3_ref[...])


def _full(shape):
    return pl.BlockSpec(shape, lambda i: tuple(0 for _ in shape))


def _bn(width):
    return pl.BlockSpec((BN, width), lambda i: (i, 0))


def _enc0_call(x, w, a_s, a_d):
    return pl.pallas_call(
        _enc0_body,
        grid=(NP // BN,),
        in_specs=[_bn(4), _full((4, H)), _full((H, 1)), _full((H, 1))],
        out_specs=[pl.BlockSpec((4, BN, HH), lambda i: (0, i, 0)),
                   _bn(1), _bn(1)],
        out_shape=[jax.ShapeDtypeStruct((4, NP, HH), jnp.float32),
                   jax.ShapeDtypeStruct((NP, 1), jnp.float32),
                   jax.ShapeDtypeStruct((NP, 1), jnp.float32)],
    )(x, w, a_s, a_d)


def _enc_call(q0, q1, q2, q3, den_a, den_b, b, w, a_s, a_d):
    return pl.pallas_call(
        _enc_body,
        grid=(NP // BN,),
        in_specs=[_bn(HH), _bn(HH), _bn(HH), _bn(HH), _bn(1), _bn(1),
                  _full((1, H)), _full((H, H)), _full((H, 1)), _full((H, 1))],
        out_specs=[pl.BlockSpec((4, BN, HH), lambda i: (0, i, 0)),
                   _bn(1), _bn(1)],
        out_shape=[jax.ShapeDtypeStruct((4, NP, HH), jnp.float32),
                   jax.ShapeDtypeStruct((NP, 1), jnp.float32),
                   jax.ShapeDtypeStruct((NP, 1), jnp.float32)],
    )(q0, q1, q2, q3, den_a, den_b, b, w, a_s, a_d)


def _heads_call(q0, q1, q2, q3, den_a, den_b, b, w1a, w1b, b1,
                vw1, vb1, vw2, vb2, lw1, lb1, lw2, lb2):
    return pl.pallas_call(
        _heads_body,
        grid=(NP // BN,),
        in_specs=[_bn(HH), _bn(HH), _bn(HH), _bn(HH), _bn(1), _bn(1),
                  _full((1, H)), _full((H, H)), _full((H, H)), _full((1, H)),
                  _full((H, 32)), _full((1, 32)), _full((32, 1)), _full((1, 1)),
                  _full((H, 32)), _full((1, 32)), _full((32, 4)), _full((1, 4))],
        out_specs=[_bn(H), _bn(H), _bn(H), _bn(1), _bn(4)],
        out_shape=[jax.ShapeDtypeStruct((NP, H), jnp.float32),
                   jax.ShapeDtypeStruct((NP, H), jnp.float32),
                   jax.ShapeDtypeStruct((NP, H), jnp.float32),
                   jax.ShapeDtypeStruct((NP, 1), jnp.float32),
                   jax.ShapeDtypeStruct((NP, 4), jnp.float32)],
    )(q0, q1, q2, q3, den_a, den_b, b, w1a, w1b, b1,
      vw1, vb1, vw2, vb2, lw1, lb1, lw2, lb2)


def _path_call(zs, zd, w2, b2, w3, b3):
    return pl.pallas_call(
        _path_body,
        grid=(E // BE,),
        in_specs=[pl.BlockSpec((BE, H), lambda i: (i, 0)),
                  pl.BlockSpec((BE, H), lambda i: (i, 0)),
                  _full((H, 32)), _full((1, 32)), _full((32, 1)), _full((1, 1))],
        out_specs=pl.BlockSpec((BE, 1), lambda i: (i, 0)),
        out_shape=jax.ShapeDtypeStruct((E, 1), jnp.float32),
    )(zs, zd, w2, b2, w3, b3)


_MESH = plsc.VectorSubcoreMesh(core_axis_name="c", subcore_axis_name="s")
_SC_PARAMS = pltpu.CompilerParams(needs_layout_passes=False,
                                  use_tc_tiling_on_sc=False)


def _w_groups(as_t, ad_t, sbuf, dbuf, wbuf, w32, cbuf, d2buf, shift,
              valid_from):
    zf16 = jnp.zeros((16,), jnp.float32)
    for g in range(CH // 16):
        row0 = g * 16
        sv = sbuf[pl.ds(row0, 16)]
        dv = dbuf[pl.ds(row0, 16)]
        a = plsc.load_gather(as_t, [sv]) + plsc.load_gather(ad_t, [dv])
        e = jnp.where(a > 0, a, 0.2 * a)
        w = jnp.exp(e - shift)
        valid = (row0 + _iota16()) < valid_from
        w = jnp.where(valid, w, 0.0)
        wbuf[pl.ds(row0, 16)] = w
        f = row0 + _iota16()
        oldcol = cbuf[pl.ds(row0, 16)]
        plsc.store_scatter(w32, [f, oldcol], zf16)
        col = jnp.bitwise_and(dv, DW - 1)
        plsc.store_scatter(w32, [f, col], w)
        cbuf[pl.ds(row0, 16)] = col
        d2buf[pl.ds(row0, 16)] = lax.shift_right_logical(dv, 5)


def _wden_body(asv, adv, src, dst, w_all, denf,
               as_t, ad_t, sbuf, dbuf, wbuf, w32, cbuf, d2buf, den_sh):
    c = lax.axis_index("c")
    t = lax.axis_index("s")
    wid = t * 2 + c

    pltpu.sync_copy(asv.at[pl.ds(0, N)], as_t)
    pltpu.sync_copy(adv.at[pl.ds(0, N)], ad_t)

    def _mxstep(i, carry):
        ma, mb = carry
        return (jnp.maximum(ma, as_t[pl.ds(i * 16, 16)]),
                jnp.maximum(mb, ad_t[pl.ds(i * 16, 16)]))

    neg = jnp.full((16,), -1e30, jnp.float32)
    ma, mb = lax.fori_loop(0, N // 16, _mxstep, (neg, neg))

    wbuf[pl.ds(0, 16)] = ma
    wbuf[pl.ds(16, 16)] = mb
    m0 = plsc.load_gather(wbuf, [jnp.zeros((16,), jnp.int32)])
    m1 = plsc.load_gather(wbuf, [jnp.full((16,), 16, jnp.int32)])
    for i in range(1, 16):
        m0 = jnp.maximum(m0, plsc.load_gather(
            wbuf, [jnp.full((16,), i, jnp.int32)]))
        m1 = jnp.maximum(m1, plsc.load_gather(
            wbuf, [jnp.full((16,), 16 + i, jnp.int32)]))
    shift = jnp.maximum(m0 + m1, jnp.zeros((16,), jnp.float32))

    zf = jnp.zeros((16,), jnp.float32)
    zi = jnp.zeros((16,), jnp.int32)
    for rr in range(CH):
        w32[rr, 0:16] = zf
        w32[rr, 16:32] = zf
    for g in range(CH // 16):
        cbuf[pl.ds(g * 16, 16)] = zi
    pltpu.sync_copy(w32.at[pl.ds(0, NDT), :],
                    den_sh.at[pl.ds(t * NDT, NDT), :])

    plsc.subcore_barrier()

    big = jnp.int32(1 << 30)
    ebase = wid * EPW

    def _echunk(k, _):
        off = ebase + k * CH
        pltpu.sync_copy(src.at[pl.ds(off, CH)], sbuf)
        pltpu.sync_copy(dst.at[pl.ds(off, CH)], dbuf)
        vfrom = jnp.minimum(jnp.maximum(E - off, 0), big)
        _w_groups(as_t, ad_t, sbuf, dbuf, wbuf, w32, cbuf, d2buf, shift,
                  vfrom)
        pltpu.sync_copy(wbuf, w_all.at[pl.ds(off, CH)])
        pltpu.sync_copy(w32, den_sh.at[d2buf], add=True)
        return 0

    lax.fori_loop(0, EPW // CH, _echunk, 0)

    sbase = wid * SELF_STRIDE

    def _schunk(k, _):
        j0 = k * CH
        for g in range(CH // 16):
            row0 = g * 16
            ids = sbase + j0 + row0 + _iota16()
            ids = jnp.where(ids < N, ids, 0)
            sbuf[pl.ds(row0, 16)] = ids
            dbuf[pl.ds(row0, 16)] = ids
        vfrom = jnp.minimum(SELF_STRIDE, N - sbase) - j0
        _w_groups(as_t, ad_t, sbuf, dbuf, wbuf, w32, cbuf, d2buf, shift,
                  vfrom)
        pltpu.sync_copy(wbuf, w_all.at[pl.ds(E_PAD + sbase + j0, CH)])
        pltpu.sync_copy(w32, den_sh.at[d2buf], add=True)
        return 0

    lax.fori_loop(0, SELF_STRIDE // CH, _schunk, 0)

    plsc.subcore_barrier()
    pltpu.sync_copy(den_sh.at[pl.ds(t * NDT, NDT), :],
                    denf.at[pl.ds(c * ND + t * NDT, NDT), :])


@functools.partial(
    pl.kernel,
    out_type=[jax.ShapeDtypeStruct((W_LEN,), jnp.float32),
              jax.ShapeDtypeStruct((2 * ND, DW), jnp.float32)],
    mesh=_MESH,
    compiler_params=_SC_PARAMS,
    scratch_types=[
        pltpu.VMEM((N,), jnp.float32),
        pltpu.VMEM((N,), jnp.float32),
        pltpu.VMEM((CH,), jnp.int32),
        pltpu.VMEM((CH,), jnp.int32),
        pltpu.VMEM((CH,), jnp.float32),
        pltpu.VMEM((CH, DW), jnp.float32),
        pltpu.VMEM((CH,), jnp.int32),
        pltpu.VMEM((CH,), jnp.int32),
        pltpu.VMEM_SHARED((ND, DW), jnp.float32),
    ],
)
def _wden_kernel(asv, adv, src, dst, w_all, denf,
                 as_t, ad_t, sbuf, dbuf, wbuf, w32, cbuf, d2buf, den_sh):
    _wden_body(asv, adv, src, dst, w_all, denf,
               as_t, ad_t, sbuf, dbuf, wbuf, w32, cbuf, d2buf, den_sh)


def _acc_body(hp4f, src, dst, w_all, accf,
              sbufL, dbufL, wbufL, sbuf, dbuf, wbuf, didx0, didx1,
              rows0, rows1, rowsS, acc_sh, gsem0, gsem1, ssem0, ssem1,
              sems):
    c = lax.axis_index("c")
    t = lax.axis_index("s")
    cN = c * NP
    r0 = t * NPT
    zf = jnp.zeros((16,), jnp.float32)
    ebase = t * EPG
    sbase = t * SPG
    SUP = 1024

    def _scale(rows, wsrc, woff):
        for g in range(CH // 16):
            wv = wsrc[pl.ds(woff + g * 16, 16)]
            for r in range(16):
                row = g * 16 + r
                wb = jnp.full((16,), wv[r], jnp.float32)
                rows[row, 0:16] = rows[row, 0:16] * wb

    for ps in (0, 1):
        pofs = 2 * ps * NP + cN

        for rr in range(CH):
            rowsS[rr, 0:16] = zf
        for j in range(NPT // CH):
            pltpu.sync_copy(rowsS, acc_sh.at[pl.ds(r0 + j * CH, CH), :])
        pltpu.sync_copy(rowsS.at[pl.ds(0, NPT % CH), :],
                        acc_sh.at[pl.ds(r0 + (NPT // CH) * CH, NPT % CH), :])

        plsc.subcore_barrier()

        def _esuper(k, _, pofs=pofs):
            offL = ebase + k * SUP
            pltpu.sync_copy(src.at[pl.ds(offL, SUP)], sbufL)
            pltpu.sync_copy(dst.at[pl.ds(offL, SUP)], dbufL)
            pltpu.sync_copy(w_all.at[pl.ds(offL, SUP)], wbufL)
            for g in range(SUP // 16):
                sbufL[pl.ds(g * 16, 16)] = sbufL[pl.ds(g * 16, 16)] + pofs

            def _pair_step(jj, _):
                a = jj * 2 * CH
                b = a + CH
                for g in range(CH // 16):
                    didx0[pl.ds(g * 16, 16)] = dbufL[pl.ds(a + g * 16, 16)]
                ga = pltpu.async_copy(hp4f.at[sbufL.at[pl.ds(a, CH)]],
                                      rows0, gsem0)
                for g in range(CH // 16):
                    didx1[pl.ds(g * 16, 16)] = dbufL[pl.ds(b + g * 16, 16)]
                gb = pltpu.async_copy(hp4f.at[sbufL.at[pl.ds(b, CH)]],
                                      rows1, gsem1)
                ga.wait()
                _scale(rows0, wbufL, a)
                sa = pltpu.async_copy(rows0, acc_sh.at[didx0], ssem0,
                                      add=True)
                gb.wait()
                _scale(rows1, wbufL, b)
                sb = pltpu.async_copy(rows1, acc_sh.at[didx1], ssem1,
                                      add=True)
                sa.wait()
                sb.wait()
                return 0

            lax.fori_loop(0, SUP // (2 * CH), _pair_step, 0)
            return 0

        lax.fori_loop(0, EPG // SUP, _esuper, 0)

        def _schunk(k, _, pofs=pofs):
            j0 = k * CH
            for g in range(CH // 16):
                row0 = g * 16
                ids = sbase + j0 + row0 + _iota16()
                ids = jnp.where(ids < N, ids, 0)
                sbuf[pl.ds(row0, 16)] = ids + pofs
                dbuf[pl.ds(row0, 16)] = ids
            pltpu.sync_copy(w_all.at[pl.ds(E_PAD + sbase + j0, CH)], wbuf)
            pltpu.async_copy(hp4f.at[sbuf], rowsS, sems).wait()
            _scale(rowsS, wbuf, 0)
            pltpu.sync_copy(rowsS, acc_sh.at[dbuf], add=True)
            return 0

        lax.fori_loop(0, SPG // CH, _schunk, 0)

        plsc.subcore_barrier()
        pltpu.sync_copy(acc_sh.at[pl.ds(r0, NPT), :],
                        accf.at[pl.ds(pofs + r0, NPT), :])


@functools.partial(
    pl.kernel,
    out_type=jax.ShapeDtypeStruct((4 * NP, HH), jnp.float32),
    mesh=_MESH,
    compiler_params=_SC_PARAMS,
    scratch_types=[
        pltpu.VMEM((1024,), jnp.int32),
        pltpu.VMEM((1024,), jnp.int32),
        pltpu.VMEM((1024,), jnp.float32),
        pltpu.VMEM((CH,), jnp.int32),
        pltpu.VMEM((CH,), jnp.int32),
        pltpu.VMEM((CH,), jnp.float32),
        pltpu.VMEM((CH,), jnp.int32),
        pltpu.VMEM((CH,), jnp.int32),
        pltpu.VMEM((CH, HH), jnp.float32),
        pltpu.VMEM((CH, HH), jnp.float32),
        pltpu.VMEM((CH, HH), jnp.float32),
        pltpu.VMEM_SHARED((NP, HH), jnp.float32),
        pltpu.SemaphoreType.DMA,
        pltpu.SemaphoreType.DMA,
        pltpu.SemaphoreType.DMA,
        pltpu.SemaphoreType.DMA,
        pltpu.SemaphoreType.DMA,
    ],
)
def _acc_kernel(hp4f, src, dst, w_all, accf,
                sbufL, dbufL, wbufL, sbuf, dbuf, wbuf, didx0, didx1,
                rows0, rows1, rowsS, acc_sh, gsem0, gsem1, ssem0, ssem1,
                sems):
    _acc_body(hp4f, src, dst, w_all, accf,
              sbufL, dbufL, wbufL, sbuf, dbuf, wbuf, didx0, didx1,
              rows0, rows1, rowsS, acc_sh, gsem0, gsem1, ssem0, ssem1,
              sems)


EPT2 = E // 32


def _pair_body(g1, g2, src, dst, zs, zd, sbuf, dbuf, rows1, rows2, sem):
    c = lax.axis_index("c")
    t = lax.axis_index("s")
    wid = t * 2 + c
    base = wid * EPT2

    def _chunk(off, nreal):
        if nreal < CH:
            zi = jnp.zeros((16,), jnp.int32)
            for g in range(CH // 16):
                sbuf[pl.ds(g * 16, 16)] = zi
                dbuf[pl.ds(g * 16, 16)] = zi
        pltpu.sync_copy(src.at[pl.ds(off, nreal)], sbuf.at[pl.ds(0, nreal)])
        pltpu.sync_copy(dst.at[pl.ds(off, nreal)], dbuf.at[pl.ds(0, nreal)])
        pltpu.async_copy(g1.at[sbuf], rows1, sem).wait()
        pltpu.async_copy(g2.at[dbuf], rows2, sem).wait()
        pltpu.sync_copy(rows1.at[pl.ds(0, nreal), :], zs.at[pl.ds(off, nreal), :])
        pltpu.sync_copy(rows2.at[pl.ds(0, nreal), :], zd.at[pl.ds(off, nreal), :])

    def _step(k, _):
        _chunk(base + k * CH, CH)
        return 0

    nfull = EPT2 // CH
    lax.fori_loop(0, nfull, _step, 0)
    _chunk(base + nfull * CH, EPT2 - nfull * CH)


@functools.partial(
    pl.kernel,
    out_type=[jax.ShapeDtypeStruct((E, H), jnp.float32),
              jax.ShapeDtypeStruct((E, H), jnp.float32)],
    mesh=_MESH,
    compiler_params=_SC_PARAMS,
    scratch_types=[
        pltpu.VMEM((CH,), jnp.int32),
        pltpu.VMEM((CH,), jnp.int32),
        pltpu.VMEM((CH, H), jnp.float32),
        pltpu.VMEM((CH, H), jnp.float32),
        pltpu.SemaphoreType.DMA,
    ],
)
def _pair_kernel(g1, g2, src, dst, zs, zd, sbuf, dbuf, rows1, rows2, sem):
    _pair_body(g1, g2, src, dst, zs, zd, sbuf, dbuf, rows1, rows2, sem)


def kernel(x, edge_index, W0, a_s0, a_d0, b0, Ws, a_ss, a_ds, bs,
           pp_W1, pp_b1, pp_W2, pp_b2, pp_W3, pp_b3,
           vp_W1, vp_b1, vp_W2, vp_b2, lp_W1, lp_b1, lp_W2, lp_b2):
    src = edge_index[0]
    dst = edge_index[1]
    pad = jnp.zeros((E_PAD - E,), jnp.int32)
    src_p = jnp.concatenate([src, pad])
    dst_p = jnp.concatenate([dst, pad])
    x_p = jnp.concatenate([x, jnp.zeros((NP - N, x.shape[1]), x.dtype)])

    hp4, sv, dv = _enc0_call(x_p, W0, a_s0.reshape(H, 1), a_d0.reshape(H, 1))

    w_xs = jnp.stack([Ws[0], Ws[1], Ws[2], Ws[2]])
    as_xs = jnp.concatenate([a_ss, a_ss[2:3]]).reshape(4, H, 1)
    ad_xs = jnp.concatenate([a_ds, a_ds[2:3]]).reshape(4, H, 1)
    b_xs = jnp.stack([b0, bs[0], bs[1], bs[2]]).reshape(4, 1, H)

    def _layer_step(carry, xs):
        hp4, sv, dv = carry[:3]
        w_all, denf = _wden_kernel(sv, dv, src_p, dst_p)
        accf = _acc_kernel(hp4.reshape(4 * NP, HH), src_p, dst_p, w_all)
        quarters = accf.reshape(4, NP, HH)
        den_a = denf[:ND].reshape(ND * DW, 1)[:NP]
        den_b = denf[ND:].reshape(ND * DW, 1)[:NP]
        hp4n, svn, dvn = _enc_call(
            quarters[0], quarters[1], quarters[2], quarters[3], den_a, den_b,
            xs["b"], xs["W"], xs["a_s"], xs["a_d"])
        return ((hp4n, svn.reshape(NP), dvn.reshape(NP), quarters,
                 den_a, den_b), None)

    init = (hp4, sv.reshape(NP), dv.reshape(NP),
            jnp.zeros((4, NP, HH), jnp.float32),
            jnp.zeros((NP, 1), jnp.float32),
            jnp.zeros((NP, 1), jnp.float32))
    carry, _ = lax.scan(
        _layer_step, init,
        {"W": w_xs, "a_s": as_xs, "a_d": ad_xs, "b": b_xs})
    quarters = carry[3]
    den_a, den_b = carry[4], carry[5]

    nf, g1, g2, vias, layers = _heads_call(
        quarters[0], quarters[1], quarters[2], quarters[3], den_a, den_b,
        bs[2].reshape(1, H),
        pp_W1[:H, :], pp_W1[H:, :], pp_b1.reshape(1, H),
        vp_W1, vp_b1.reshape(1, 32), vp_W2, vp_b2.reshape(1, 1),
        lp_W1, lp_b1.reshape(1, 32), lp_W2, lp_b2.reshape(1, 4))

    zs, zd = _pair_kernel(g1, g2, src, dst)
    paths = _path_call(zs, zd, pp_W2, pp_b2.reshape(1, 32),
                       pp_W3, pp_b3.reshape(1, 1))

    return (paths.reshape(E), vias.reshape(NP)[:N], layers[:N], nf[:N])


# speedup vs baseline: 17.2109x; 1.0744x over previous
import functools

import jax
import jax.numpy as jnp
from jax import lax
from jax.experimental import pallas as pl
from jax.experimental.pallas import tpu as pltpu
from jax.experimental.pallas import tpu_sc as plsc

N = 50000
NP = 50048
E = 800000
H = 64
HH = 16
DW = 32
CH = 128
NSUB = 16
E_PAD = 802816
EPW = E_PAD // 32
SELF_STRIDE = 1664
SELF_LEN = 32 * SELF_STRIDE
W_LEN = E_PAD + SELF_LEN
EPG = E_PAD // NSUB
SPG = SELF_LEN // NSUB
NPT = NP // NSUB
ND = 1664
NDT = ND // NSUB
BN = 3128
BE = 8000


def _iota16():
    return lax.iota(jnp.int32, 16)


def _enc0_body(x_ref, w_ref, as_ref, ad_ref, hp4_ref, sv_ref, dv_ref):
    hp = jnp.dot(x_ref[...], w_ref[...], preferred_element_type=jnp.float32)
    for q in range(4):
        hp4_ref[q] = hp[:, q * HH:(q + 1) * HH]
    sv_ref[...] = jnp.dot(hp, as_ref[...], preferred_element_type=jnp.float32)
    dv_ref[...] = jnp.dot(hp, ad_ref[...], preferred_element_type=jnp.float32)


def _enc_body(q0_ref, q1_ref, q2_ref, q3_ref, dena_ref, denb_ref, b_ref,
              w_ref, as_ref, ad_ref, hp4_ref, sv_ref, dv_ref):
    d = dena_ref[...] + denb_ref[...]
    b = b_ref[...]
    w = w_ref[...]
    hp = None
    for q, qref in enumerate((q0_ref, q1_ref, q2_ref, q3_ref)):
        hq = jax.nn.relu(qref[...] / d + b[:, q * HH:(q + 1) * HH])
        part = jnp.dot(hq, w[q * HH:(q + 1) * HH, :],
                       preferred_element_type=jnp.float32)
        hp = part if hp is None else hp + part
    for q in range(4):
        hp4_ref[q] = hp[:, q * HH:(q + 1) * HH]
    sv_ref[...] = jnp.dot(hp, as_ref[...], preferred_element_type=jnp.float32)
    dv_ref[...] = jnp.dot(hp, ad_ref[...], preferred_element_type=jnp.float32)


def _sigmoid(x):
    return 1.0 / (1.0 + jnp.exp(-x))


def _heads_body(q0_ref, q1_ref, q2_ref, q3_ref, dena_ref, denb_ref, b_ref,
                w1a_ref, w1b_ref, b1_ref,
                vw1_ref, vb1_ref, vw2_ref, vb2_ref,
                lw1_ref, lb1_ref, lw2_ref, lb2_ref,
                nf_ref, g1_ref, g2_ref, via_ref, lay_ref):
    d = dena_ref[...] + denb_ref[...]
    b = b_ref[...]
    nf = jnp.concatenate(
        [qref[...] / d + b[:, q * HH:(q + 1) * HH]
         for q, qref in enumerate((q0_ref, q1_ref, q2_ref, q3_ref))], axis=-1)
    nf_ref[...] = nf
    g1_ref[...] = jnp.dot(nf, w1a_ref[...], preferred_element_type=jnp.float32) + b1_ref[...]
    g2_ref[...] = jnp.dot(nf, w1b_ref[...], preferred_element_type=jnp.float32)
    v = jax.nn.relu(jnp.dot(nf, vw1_ref[...], preferred_element_type=jnp.float32) + vb1_ref[...])
    via_ref[...] = _sigmoid(jnp.dot(v, vw2_ref[...], preferred_element_type=jnp.float32) + vb2_ref[...])
    l = jax.nn.relu(jnp.dot(nf, lw1_ref[...], preferred_element_type=jnp.float32) + lb1_ref[...])
    lg = jnp.dot(l, lw2_ref[...], preferred_element_type=jnp.float32) + lb2_ref[...]
    m = jnp.max(lg, axis=-1, keepdims=True)
    p = jnp.exp(lg - m)
    lay_ref[...] = p / jnp.sum(p, axis=-1, keepdims=True)


def _path_body(zs_ref, zd_ref, w2_ref, b2_ref, w3_ref, b3_ref, out_ref):
    p = jax.nn.relu(zs_ref[...] + zd_ref[...])
    p2 = jax.nn.relu(jnp.dot(p, w2_ref[...], preferred_element_type=jnp.float32) + b2_ref[...])
    out_ref[...] = _sigmoid(jnp.dot(p2, w3_ref[...], preferred_element_type=jnp.float32) + b3_ref[...])


def _full(shape):
    return pl.BlockSpec(shape, lambda i: tuple(0 for _ in shape))


def _bn(width):
    return pl.BlockSpec((BN, width), lambda i: (i, 0))


def _enc0_call(x, w, a_s, a_d):
    return pl.pallas_call(
        _enc0_body,
        grid=(NP // BN,),
        in_specs=[_bn(4), _full((4, H)), _full((H, 1)), _full((H, 1))],
        out_specs=[pl.BlockSpec((4, BN, HH), lambda i: (0, i, 0)),
                   _bn(1), _bn(1)],
        out_shape=[jax.ShapeDtypeStruct((4, NP, HH), jnp.float32),
                   jax.ShapeDtypeStruct((NP, 1), jnp.float32),
                   jax.ShapeDtypeStruct((NP, 1), jnp.float32)],
    )(x, w, a_s, a_d)


def _enc_call(q0, q1, q2, q3, den_a, den_b, b, w, a_s, a_d):
    return pl.pallas_call(
        _enc_body,
        grid=(NP // BN,),
        in_specs=[_bn(HH), _bn(HH), _bn(HH), _bn(HH), _bn(1), _bn(1),
                  _full((1, H)), _full((H, H)), _full((H, 1)), _full((H, 1))],
        out_specs=[pl.BlockSpec((4, BN, HH), lambda i: (0, i, 0)),
                   _bn(1), _bn(1)],
        out_shape=[jax.ShapeDtypeStruct((4, NP, HH), jnp.float32),
                   jax.ShapeDtypeStruct((NP, 1), jnp.float32),
                   jax.ShapeDtypeStruct((NP, 1), jnp.float32)],
    )(q0, q1, q2, q3, den_a, den_b, b, w, a_s, a_d)


def _heads_call(q0, q1, q2, q3, den_a, den_b, b, w1a, w1b, b1,
                vw1, vb1, vw2, vb2, lw1, lb1, lw2, lb2):
    return pl.pallas_call(
        _heads_body,
        grid=(NP // BN,),
        in_specs=[_bn(HH), _bn(HH), _bn(HH), _bn(HH), _bn(1), _bn(1),
                  _full((1, H)), _full((H, H)), _full((H, H)), _full((1, H)),
                  _full((H, 32)), _full((1, 32)), _full((32, 1)), _full((1, 1)),
                  _full((H, 32)), _full((1, 32)), _full((32, 4)), _full((1, 4))],
        out_specs=[_bn(H), _bn(H), _bn(H), _bn(1), _bn(4)],
        out_shape=[jax.ShapeDtypeStruct((NP, H), jnp.float32),
                   jax.ShapeDtypeStruct((NP, H), jnp.float32),
                   jax.ShapeDtypeStruct((NP, H), jnp.float32),
                   jax.ShapeDtypeStruct((NP, 1), jnp.float32),
                   jax.ShapeDtypeStruct((NP, 4), jnp.float32)],
    )(q0, q1, q2, q3, den_a, den_b, b, w1a, w1b, b1,
      vw1, vb1, vw2, vb2, lw1, lb1, lw2, lb2)


def _path_call(zs, zd, w2, b2, w3, b3):
    return pl.pallas_call(
        _path_body,
        grid=(E // BE,),
        in_specs=[pl.BlockSpec((BE, H), lambda i: (i, 0)),
                  pl.BlockSpec((BE, H), lambda i: (i, 0)),
                  _full((H, 32)), _full((1, 32)), _full((32, 1)), _full((1, 1))],
        out_specs=pl.BlockSpec((BE, 1), lambda i: (i, 0)),
        out_shape=jax.ShapeDtypeStruct((E, 1), jnp.float32),
    )(zs, zd, w2, b2, w3, b3)


_MESH = plsc.VectorSubcoreMesh(core_axis_name="c", subcore_axis_name="s")
_SC_PARAMS = pltpu.CompilerParams(needs_layout_passes=False,
                                  use_tc_tiling_on_sc=False)


def _w_groups(as_t, ad_t, sref, dref, wref, w32, cbuf, d2buf, shift,
              valid_from, off):
    zf16 = jnp.zeros((16,), jnp.float32)
    for g in range(CH // 16):
        row0 = g * 16
        sv = sref[pl.ds(off + row0, 16)]
        dv = dref[pl.ds(off + row0, 16)]
        a = plsc.load_gather(as_t, [sv]) + plsc.load_gather(ad_t, [dv])
        e = jnp.where(a > 0, a, 0.2 * a)
        w = jnp.exp(e - shift)
        valid = (row0 + _iota16()) < valid_from
        w = jnp.where(valid, w, 0.0)
        wref[pl.ds(off + row0, 16)] = w
        f = row0 + _iota16()
        oldcol = cbuf[pl.ds(row0, 16)]
        plsc.store_scatter(w32, [f, oldcol], zf16)
        col = jnp.bitwise_and(dv, DW - 1)
        plsc.store_scatter(w32, [f, col], w)
        cbuf[pl.ds(row0, 16)] = col
        d2buf[pl.ds(row0, 16)] = lax.shift_right_logical(dv, 5)


def _wden_body(asv, adv, src, dst, w_all, denf,
               as_t, ad_t, sbufL, dbufL, wbufL, sbuf, dbuf, wbuf, w32,
               cbuf, d2buf, den_sh):
    c = lax.axis_index("c")
    t = lax.axis_index("s")
    wid = t * 2 + c

    pltpu.sync_copy(asv.at[pl.ds(0, N)], as_t)
    pltpu.sync_copy(adv.at[pl.ds(0, N)], ad_t)

    def _mxstep(i, carry):
        ma, mb = carry
        return (jnp.maximum(ma, as_t[pl.ds(i * 16, 16)]),
                jnp.maximum(mb, ad_t[pl.ds(i * 16, 16)]))

    neg = jnp.full((16,), -1e30, jnp.float32)
    ma, mb = lax.fori_loop(0, N // 16, _mxstep, (neg, neg))

    wbuf[pl.ds(0, 16)] = ma
    wbuf[pl.ds(16, 16)] = mb
    m0 = plsc.load_gather(wbuf, [jnp.zeros((16,), jnp.int32)])
    m1 = plsc.load_gather(wbuf, [jnp.full((16,), 16, jnp.int32)])
    for i in range(1, 16):
        m0 = jnp.maximum(m0, plsc.load_gather(
            wbuf, [jnp.full((16,), i, jnp.int32)]))
        m1 = jnp.maximum(m1, plsc.load_gather(
            wbuf, [jnp.full((16,), 16 + i, jnp.int32)]))
    shift = jnp.maximum(m0 + m1, jnp.zeros((16,), jnp.float32))

    zf = jnp.zeros((16,), jnp.float32)
    zi = jnp.zeros((16,), jnp.int32)
    for rr in range(CH):
        w32[rr, 0:16] = zf
        w32[rr, 16:32] = zf
    for g in range(CH // 16):
        cbuf[pl.ds(g * 16, 16)] = zi
    pltpu.sync_copy(w32.at[pl.ds(0, NDT), :],
                    den_sh.at[pl.ds(t * NDT, NDT), :])

    plsc.subcore_barrier()

    big = jnp.int32(1 << 30)
    ebase = wid * EPW
    SUPW = 512

    def _esuper(k, _):
        offL = ebase + k * SUPW
        pltpu.sync_copy(src.at[pl.ds(offL, SUPW)], sbufL)
        pltpu.sync_copy(dst.at[pl.ds(offL, SUPW)], dbufL)
        for j in range(SUPW // CH):
            off = j * CH
            vfrom = jnp.minimum(jnp.maximum(E - (offL + off), 0), big)
            _w_groups(as_t, ad_t, sbufL, dbufL, wbufL, w32, cbuf, d2buf,
                      shift, vfrom, off)
            pltpu.sync_copy(w32, den_sh.at[d2buf], add=True)
        pltpu.sync_copy(wbufL, w_all.at[pl.ds(offL, SUPW)])
        return 0

    lax.fori_loop(0, EPW // SUPW, _esuper, 0)

    sbase = wid * SELF_STRIDE

    def _schunk(k, _):
        j0 = k * CH
        for g in range(CH // 16):
            row0 = g * 16
            ids = sbase + j0 + row0 + _iota16()
            ids = jnp.where(ids < N, ids, 0)
            sbuf[pl.ds(row0, 16)] = ids
            dbuf[pl.ds(row0, 16)] = ids
        vfrom = jnp.minimum(SELF_STRIDE, N - sbase) - j0
        _w_groups(as_t, ad_t, sbuf, dbuf, wbuf, w32, cbuf, d2buf, shift,
                  vfrom, 0)
        pltpu.sync_copy(wbuf, w_all.at[pl.ds(E_PAD + sbase + j0, CH)])
        pltpu.sync_copy(w32, den_sh.at[d2buf], add=True)
        return 0

    lax.fori_loop(0, SELF_STRIDE // CH, _schunk, 0)

    plsc.subcore_barrier()
    pltpu.sync_copy(den_sh.at[pl.ds(t * NDT, NDT), :],
                    denf.at[pl.ds(c * ND + t * NDT, NDT), :])


@functools.partial(
    pl.kernel,
    out_type=[jax.ShapeDtypeStruct((W_LEN,), jnp.float32),
              jax.ShapeDtypeStruct((2 * ND, DW), jnp.float32)],
    mesh=_MESH,
    compiler_params=_SC_PARAMS,
    scratch_types=[
        pltpu.VMEM((N,), jnp.float32),
        pltpu.VMEM((N,), jnp.float32),
        pltpu.VMEM((512,), jnp.int32),
        pltpu.VMEM((512,), jnp.int32),
        pltpu.VMEM((512,), jnp.float32),
        pltpu.VMEM((CH,), jnp.int32),
        pltpu.VMEM((CH,), jnp.int32),
        pltpu.VMEM((CH,), jnp.float32),
        pltpu.VMEM((CH, DW), jnp.float32),
        pltpu.VMEM((CH,), jnp.int32),
        pltpu.VMEM((CH,), jnp.int32),
        pltpu.VMEM_SHARED((ND, DW), jnp.float32),
    ],
)
def _wden_kernel(asv, adv, src, dst, w_all, denf,
                 as_t, ad_t, sbufL, dbufL, wbufL, sbuf, dbuf, wbuf, w32,
                 cbuf, d2buf, den_sh):
    _wden_body(asv, adv, src, dst, w_all, denf,
               as_t, ad_t, sbufL, dbufL, wbufL, sbuf, dbuf, wbuf, w32,
               cbuf, d2buf, den_sh)


def _acc_body(hp4f, src, dst, w_all, accf,
              sbufL, dbufL, wbufL, sbuf, dbuf, wbuf, didx0, didx1,
              rows0, rows1, rowsS, acc_sh, gsem0, gsem1, ssem0, ssem1,
              sems):
    c = lax.axis_index("c")
    t = lax.axis_index("s")
    cN = c * NP
    r0 = t * NPT
    zf = jnp.zeros((16,), jnp.float32)
    ebase = t * EPG
    sbase = t * SPG
    SUP = 1024

    def _scale(rows, wsrc, woff):
        for g in range(CH // 16):
            wv = wsrc[pl.ds(woff + g * 16, 16)]
            for r in range(16):
                row = g * 16 + r
                wb = jnp.full((16,), wv[r], jnp.float32)
                rows[row, 0:16] = rows[row, 0:16] * wb

    for ps in (0, 1):
        pofs = 2 * ps * NP + cN

        for rr in range(CH):
            rowsS[rr, 0:16] = zf
        for j in range(NPT // CH):
            pltpu.sync_copy(rowsS, acc_sh.at[pl.ds(r0 + j * CH, CH), :])
        pltpu.sync_copy(rowsS.at[pl.ds(0, NPT % CH), :],
                        acc_sh.at[pl.ds(r0 + (NPT // CH) * CH, NPT % CH), :])

        plsc.subcore_barrier()

        def _esuper(k, _, pofs=pofs):
            offL = ebase + k * SUP
            pltpu.sync_copy(src.at[pl.ds(offL, SUP)], sbufL)
            pltpu.sync_copy(dst.at[pl.ds(offL, SUP)], dbufL)
            pltpu.sync_copy(w_all.at[pl.ds(offL, SUP)], wbufL)
            for g in range(SUP // 16):
                sbufL[pl.ds(g * 16, 16)] = sbufL[pl.ds(g * 16, 16)] + pofs

            def _pair_step(jj, _):
                a = jj * 2 * CH
                b = a + CH
                for g in range(CH // 16):
                    didx0[pl.ds(g * 16, 16)] = dbufL[pl.ds(a + g * 16, 16)]
                ga = pltpu.async_copy(hp4f.at[sbufL.at[pl.ds(a, CH)]],
                                      rows0, gsem0)
                for g in range(CH // 16):
                    didx1[pl.ds(g * 16, 16)] = dbufL[pl.ds(b + g * 16, 16)]
                gb = pltpu.async_copy(hp4f.at[sbufL.at[pl.ds(b, CH)]],
                                      rows1, gsem1)
                ga.wait()
                _scale(rows0, wbufL, a)
                sa = pltpu.async_copy(rows0, acc_sh.at[didx0], ssem0,
                                      add=True)
                gb.wait()
                _scale(rows1, wbufL, b)
                sb = pltpu.async_copy(rows1, acc_sh.at[didx1], ssem1,
                                      add=True)
                sa.wait()
                sb.wait()
                return 0

            lax.fori_loop(0, SUP // (2 * CH), _pair_step, 0)
            return 0

        lax.fori_loop(0, EPG // SUP, _esuper, 0)

        def _schunk(k, _, pofs=pofs):
            j0 = k * CH
            for g in range(CH // 16):
                row0 = g * 16
                ids = sbase + j0 + row0 + _iota16()
                ids = jnp.where(ids < N, ids, 0)
                sbuf[pl.ds(row0, 16)] = ids + pofs
                dbuf[pl.ds(row0, 16)] = ids
            pltpu.sync_copy(w_all.at[pl.ds(E_PAD + sbase + j0, CH)], wbuf)
            pltpu.async_copy(hp4f.at[sbuf], rowsS, sems).wait()
            _scale(rowsS, wbuf, 0)
            pltpu.sync_copy(rowsS, acc_sh.at[dbuf], add=True)
            return 0

        lax.fori_loop(0, SPG // CH, _schunk, 0)

        plsc.subcore_barrier()
        pltpu.sync_copy(acc_sh.at[pl.ds(r0, NPT), :],
                        accf.at[pl.ds(pofs + r0, NPT), :])


@functools.partial(
    pl.kernel,
    out_type=jax.ShapeDtypeStruct((4 * NP, HH), jnp.float32),
    mesh=_MESH,
    compiler_params=_SC_PARAMS,
    scratch_types=[
        pltpu.VMEM((1024,), jnp.int32),
        pltpu.VMEM((1024,), jnp.int32),
        pltpu.VMEM((1024,), jnp.float32),
        pltpu.VMEM((CH,), jnp.int32),
        pltpu.VMEM((CH,), jnp.int32),
        pltpu.VMEM((CH,), jnp.float32),
        pltpu.VMEM((CH,), jnp.int32),
        pltpu.VMEM((CH,), jnp.int32),
        pltpu.VMEM((CH, HH), jnp.float32),
        pltpu.VMEM((CH, HH), jnp.float32),
        pltpu.VMEM((CH, HH), jnp.float32),
        pltpu.VMEM_SHARED((NP, HH), jnp.float32),
        pltpu.SemaphoreType.DMA,
        pltpu.SemaphoreType.DMA,
        pltpu.SemaphoreType.DMA,
        pltpu.SemaphoreType.DMA,
        pltpu.SemaphoreType.DMA,
    ],
)
def _acc_kernel(hp4f, src, dst, w_all, accf,
                sbufL, dbufL, wbufL, sbuf, dbuf, wbuf, didx0, didx1,
                rows0, rows1, rowsS, acc_sh, gsem0, gsem1, ssem0, ssem1,
                sems):
    _acc_body(hp4f, src, dst, w_all, accf,
              sbufL, dbufL, wbufL, sbuf, dbuf, wbuf, didx0, didx1,
              rows0, rows1, rowsS, acc_sh, gsem0, gsem1, ssem0, ssem1,
              sems)


EPT2 = E // 32


def _pair_body(g1, g2, src, dst, zs, zd, sbuf, dbuf, rows1, rows2, sem):
    c = lax.axis_index("c")
    t = lax.axis_index("s")
    wid = t * 2 + c
    base = wid * EPT2

    def _chunk(off, nreal):
        if nreal < CH:
            zi = jnp.zeros((16,), jnp.int32)
            for g in range(CH // 16):
                sbuf[pl.ds(g * 16, 16)] = zi
                dbuf[pl.ds(g * 16, 16)] = zi
        pltpu.sync_copy(src.at[pl.ds(off, nreal)], sbuf.at[pl.ds(0, nreal)])
        pltpu.sync_copy(dst.at[pl.ds(off, nreal)], dbuf.at[pl.ds(0, nreal)])
        pltpu.async_copy(g1.at[sbuf], rows1, sem).wait()
        pltpu.async_copy(g2.at[dbuf], rows2, sem).wait()
        pltpu.sync_copy(rows1.at[pl.ds(0, nreal), :], zs.at[pl.ds(off, nreal), :])
        pltpu.sync_copy(rows2.at[pl.ds(0, nreal), :], zd.at[pl.ds(off, nreal), :])

    def _step(k, _):
        _chunk(base + k * CH, CH)
        return 0

    nfull = EPT2 // CH
    lax.fori_loop(0, nfull, _step, 0)
    _chunk(base + nfull * CH, EPT2 - nfull * CH)


@functools.partial(
    pl.kernel,
    out_type=[jax.ShapeDtypeStruct((E, H), jnp.float32),
              jax.ShapeDtypeStruct((E, H), jnp.float32)],
    mesh=_MESH,
    compiler_params=_SC_PARAMS,
    scratch_types=[
        pltpu.VMEM((CH,), jnp.int32),
        pltpu.VMEM((CH,), jnp.int32),
        pltpu.VMEM((CH, H), jnp.float32),
        pltpu.VMEM((CH, H), jnp.float32),
        pltpu.SemaphoreType.DMA,
    ],
)
def _pair_kernel(g1, g2, src, dst, zs, zd, sbuf, dbuf, rows1, rows2, sem):
    _pair_body(g1, g2, src, dst, zs, zd, sbuf, dbuf, rows1, rows2, sem)


def kernel(x, edge_index, W0, a_s0, a_d0, b0, Ws, a_ss, a_ds, bs,
           pp_W1, pp_b1, pp_W2, pp_b2, pp_W3, pp_b3,
           vp_W1, vp_b1, vp_W2, vp_b2, lp_W1, lp_b1, lp_W2, lp_b2):
    src = edge_index[0]
    dst = edge_index[1]
    pad = jnp.zeros((E_PAD - E,), jnp.int32)
    src_p = jnp.concatenate([src, pad])
    dst_p = jnp.concatenate([dst, pad])
    x_p = jnp.concatenate([x, jnp.zeros((NP - N, x.shape[1]), x.dtype)])

    hp4, sv, dv = _enc0_call(x_p, W0, a_s0.reshape(H, 1), a_d0.reshape(H, 1))

    w_xs = jnp.stack([Ws[0], Ws[1], Ws[2], Ws[2]])
    as_xs = jnp.concatenate([a_ss, a_ss[2:3]]).reshape(4, H, 1)
    ad_xs = jnp.concatenate([a_ds, a_ds[2:3]]).reshape(4, H, 1)
    b_xs = jnp.stack([b0, bs[0], bs[1], bs[2]]).reshape(4, 1, H)

    def _layer_step(carry, xs):
        hp4, sv, dv = carry[:3]
        w_all, denf = _wden_kernel(sv, dv, src_p, dst_p)
        accf = _acc_kernel(hp4.reshape(4 * NP, HH), src_p, dst_p, w_all)
        quarters = accf.reshape(4, NP, HH)
        den_a = denf[:ND].reshape(ND * DW, 1)[:NP]
        den_b = denf[ND:].reshape(ND * DW, 1)[:NP]
        hp4n, svn, dvn = _enc_call(
            quarters[0], quarters[1], quarters[2], quarters[3], den_a, den_b,
            xs["b"], xs["W"], xs["a_s"], xs["a_d"])
        return ((hp4n, svn.reshape(NP), dvn.reshape(NP), quarters,
                 den_a, den_b), None)

    init = (hp4, sv.reshape(NP), dv.reshape(NP),
            jnp.zeros((4, NP, HH), jnp.float32),
            jnp.zeros((NP, 1), jnp.float32),
            jnp.zeros((NP, 1), jnp.float32))
    carry, _ = lax.scan(
        _layer_step, init,
        {"W": w_xs, "a_s": as_xs, "a_d": ad_xs, "b": b_xs})
    quarters = carry[3]
    den_a, den_b = carry[4], carry[5]

    nf, g1, g2, vias, layers = _heads_call(
        quarters[0], quarters[1], quarters[2], quarters[3], den_a, den_b,
        bs[2].reshape(1, H),
        pp_W1[:H, :], pp_W1[H:, :], pp_b1.reshape(1, H),
        vp_W1, vp_b1.reshape(1, 32), vp_W2, vp_b2.reshape(1, 1),
        lp_W1, lp_b1.reshape(1, 32), lp_W2, lp_b2.reshape(1, 4))

    zs, zd = _pair_kernel(g1, g2, src, dst)
    paths = _path_call(zs, zd, pp_W2, pp_b2.reshape(1, 32),
                       pp_W3, pp_b3.reshape(1, 1))

    return (paths.reshape(E), vias.reshape(NP)[:N], layers[:N], nf[:N])


# speedup vs baseline: 18.0449x; 1.0485x over previous
import functools

import jax
import jax.numpy as jnp
from jax import lax
from jax.experimental import pallas as pl
from jax.experimental.pallas import tpu as pltpu
from jax.experimental.pallas import tpu_sc as plsc

N = 50000
NP = 50048
E = 800000
H = 64
HH = 16
DW = 32
CH = 128
NSUB = 16
E_PAD = 802816
EPW = E_PAD // 32
SELF_STRIDE = 1664
SELF_LEN = 32 * SELF_STRIDE
W_LEN = E_PAD + SELF_LEN
EPG = E_PAD // NSUB
SPG = SELF_LEN // NSUB
NPT = NP // NSUB
ND = 1664
NDT = ND // NSUB
BN = 3128
BE = 8000


def _iota16():
    return lax.iota(jnp.int32, 16)


def _enc0_body(x_ref, w_ref, as_ref, ad_ref, hp4_ref, sv_ref, dv_ref):
    hp = jnp.dot(x_ref[...], w_ref[...], preferred_element_type=jnp.float32)
    for q in range(4):
        hp4_ref[q] = hp[:, q * HH:(q + 1) * HH]
    sv_ref[...] = jnp.dot(hp, as_ref[...], preferred_element_type=jnp.float32)
    dv_ref[...] = jnp.dot(hp, ad_ref[...], preferred_element_type=jnp.float32)


def _enc_body(q0_ref, q1_ref, q2_ref, q3_ref, dena_ref, denb_ref, b_ref,
              w_ref, as_ref, ad_ref, hp4_ref, sv_ref, dv_ref):
    d = dena_ref[...] + denb_ref[...]
    b = b_ref[...]
    w = w_ref[...]
    hp = None
    for q, qref in enumerate((q0_ref, q1_ref, q2_ref, q3_ref)):
        hq = jax.nn.relu(qref[...] / d + b[:, q * HH:(q + 1) * HH])
        part = jnp.dot(hq, w[q * HH:(q + 1) * HH, :],
                       preferred_element_type=jnp.float32)
        hp = part if hp is None else hp + part
    for q in range(4):
        hp4_ref[q] = hp[:, q * HH:(q + 1) * HH]
    sv_ref[...] = jnp.dot(hp, as_ref[...], preferred_element_type=jnp.float32)
    dv_ref[...] = jnp.dot(hp, ad_ref[...], preferred_element_type=jnp.float32)


def _sigmoid(x):
    return 1.0 / (1.0 + jnp.exp(-x))


def _heads_body(q0_ref, q1_ref, q2_ref, q3_ref, dena_ref, denb_ref, b_ref,
                w1a_ref, w1b_ref, b1_ref,
                vw1_ref, vb1_ref, vw2_ref, vb2_ref,
                lw1_ref, lb1_ref, lw2_ref, lb2_ref,
                nf_ref, g1_ref, g2_ref, via_ref, lay_ref):
    d = dena_ref[...] + denb_ref[...]
    b = b_ref[...]
    nf = jnp.concatenate(
        [qref[...] / d + b[:, q * HH:(q + 1) * HH]
         for q, qref in enumerate((q0_ref, q1_ref, q2_ref, q3_ref))], axis=-1)
    nf_ref[...] = nf
    g1_ref[...] = jnp.dot(nf, w1a_ref[...], preferred_element_type=jnp.float32) + b1_ref[...]
    g2_ref[...] = jnp.dot(nf, w1b_ref[...], preferred_element_type=jnp.float32)
    v = jax.nn.relu(jnp.dot(nf, vw1_ref[...], preferred_element_type=jnp.float32) + vb1_ref[...])
    via_ref[...] = _sigmoid(jnp.dot(v, vw2_ref[...], preferred_element_type=jnp.float32) + vb2_ref[...])
    l = jax.nn.relu(jnp.dot(nf, lw1_ref[...], preferred_element_type=jnp.float32) + lb1_ref[...])
    lg = jnp.dot(l, lw2_ref[...], preferred_element_type=jnp.float32) + lb2_ref[...]
    m = jnp.max(lg, axis=-1, keepdims=True)
    p = jnp.exp(lg - m)
    lay_ref[...] = p / jnp.sum(p, axis=-1, keepdims=True)


def _path_body(zs_ref, zd_ref, w2_ref, b2_ref, w3_ref, b3_ref, out_ref):
    p = jax.nn.relu(zs_ref[...] + zd_ref[...])
    p2 = jax.nn.relu(jnp.dot(p, w2_ref[...], preferred_element_type=jnp.float32) + b2_ref[...])
    out_ref[...] = _sigmoid(jnp.dot(p2, w3_ref[...], preferred_element_type=jnp.float32) + b3_ref[...])


def _full(shape):
    return pl.BlockSpec(shape, lambda i: tuple(0 for _ in shape))


def _bn(width):
    return pl.BlockSpec((BN, width), lambda i: (i, 0))


def _enc0_call(x, w, a_s, a_d):
    return pl.pallas_call(
        _enc0_body,
        grid=(NP // BN,),
        in_specs=[_bn(4), _full((4, H)), _full((H, 1)), _full((H, 1))],
        out_specs=[pl.BlockSpec((4, BN, HH), lambda i: (0, i, 0)),
                   _bn(1), _bn(1)],
        out_shape=[jax.ShapeDtypeStruct((4, NP, HH), jnp.float32),
                   jax.ShapeDtypeStruct((NP, 1), jnp.float32),
                   jax.ShapeDtypeStruct((NP, 1), jnp.float32)],
    )(x, w, a_s, a_d)


def _enc_call(q0, q1, q2, q3, den_a, den_b, b, w, a_s, a_d):
    return pl.pallas_call(
        _enc_body,
        grid=(NP // BN,),
        in_specs=[_bn(HH), _bn(HH), _bn(HH), _bn(HH), _bn(1), _bn(1),
                  _full((1, H)), _full((H, H)), _full((H, 1)), _full((H, 1))],
        out_specs=[pl.BlockSpec((4, BN, HH), lambda i: (0, i, 0)),
                   _bn(1), _bn(1)],
        out_shape=[jax.ShapeDtypeStruct((4, NP, HH), jnp.float32),
                   jax.ShapeDtypeStruct((NP, 1), jnp.float32),
                   jax.ShapeDtypeStruct((NP, 1), jnp.float32)],
    )(q0, q1, q2, q3, den_a, den_b, b, w, a_s, a_d)


def _heads_call(q0, q1, q2, q3, den_a, den_b, b, w1a, w1b, b1,
                vw1, vb1, vw2, vb2, lw1, lb1, lw2, lb2):
    return pl.pallas_call(
        _heads_body,
        grid=(NP // BN,),
        in_specs=[_bn(HH), _bn(HH), _bn(HH), _bn(HH), _bn(1), _bn(1),
                  _full((1, H)), _full((H, H)), _full((H, H)), _full((1, H)),
                  _full((H, 32)), _full((1, 32)), _full((32, 1)), _full((1, 1)),
                  _full((H, 32)), _full((1, 32)), _full((32, 4)), _full((1, 4))],
        out_specs=[_bn(H), _bn(H), _bn(H), _bn(1), _bn(4)],
        out_shape=[jax.ShapeDtypeStruct((NP, H), jnp.float32),
                   jax.ShapeDtypeStruct((NP, H), jnp.float32),
                   jax.ShapeDtypeStruct((NP, H), jnp.float32),
                   jax.ShapeDtypeStruct((NP, 1), jnp.float32),
                   jax.ShapeDtypeStruct((NP, 4), jnp.float32)],
    )(q0, q1, q2, q3, den_a, den_b, b, w1a, w1b, b1,
      vw1, vb1, vw2, vb2, lw1, lb1, lw2, lb2)


def _path_call(zs, zd, w2, b2, w3, b3):
    return pl.pallas_call(
        _path_body,
        grid=(E // BE,),
        in_specs=[pl.BlockSpec((BE, H), lambda i: (i, 0)),
                  pl.BlockSpec((BE, H), lambda i: (i, 0)),
                  _full((H, 32)), _full((1, 32)), _full((32, 1)), _full((1, 1))],
        out_specs=pl.BlockSpec((BE, 1), lambda i: (i, 0)),
        out_shape=jax.ShapeDtypeStruct((E, 1), jnp.float32),
    )(zs, zd, w2, b2, w3, b3)


_MESH = plsc.VectorSubcoreMesh(core_axis_name="c", subcore_axis_name="s")
_SC_PARAMS = pltpu.CompilerParams(needs_layout_passes=False,
                                  use_tc_tiling_on_sc=False)


def _w_groups(as_t, ad_t, sref, dref, wref, w32, cbuf, d2buf, shift,
              valid_from, off):
    zf16 = jnp.zeros((16,), jnp.float32)
    for g in range(CH // 16):
        row0 = g * 16
        sv = sref[pl.ds(off + row0, 16)]
        dv = dref[pl.ds(off + row0, 16)]
        a = plsc.load_gather(as_t, [sv]) + plsc.load_gather(ad_t, [dv])
        e = jnp.where(a > 0, a, 0.2 * a)
        w = jnp.exp(e - shift)
        valid = (row0 + _iota16()) < valid_from
        w = jnp.where(valid, w, 0.0)
        wref[pl.ds(off + row0, 16)] = w
        f = row0 + _iota16()
        oldcol = cbuf[pl.ds(row0, 16)]
        plsc.store_scatter(w32, [f, oldcol], zf16)
        col = jnp.bitwise_and(dv, DW - 1)
        plsc.store_scatter(w32, [f, col], w)
        cbuf[pl.ds(row0, 16)] = col
        d2buf[pl.ds(row0, 16)] = lax.shift_right_logical(dv, 5)


def _wden_body(asv, adv, src, dst, w_all, denf,
               as_t, ad_t, sbufL, dbufL, wbufL, sbuf, dbuf, wbuf, w32,
               cbuf, d2buf, den_sh):
    c = lax.axis_index("c")
    t = lax.axis_index("s")
    wid = t * 2 + c

    pltpu.sync_copy(asv.at[pl.ds(0, N)], as_t)
    pltpu.sync_copy(adv.at[pl.ds(0, N)], ad_t)

    def _mxstep(i, carry):
        ma, mb = carry
        return (jnp.maximum(ma, as_t[pl.ds(i * 16, 16)]),
                jnp.maximum(mb, ad_t[pl.ds(i * 16, 16)]))

    neg = jnp.full((16,), -1e30, jnp.float32)
    ma, mb = lax.fori_loop(0, N // 16, _mxstep, (neg, neg))

    wbuf[pl.ds(0, 16)] = ma
    wbuf[pl.ds(16, 16)] = mb
    m0 = plsc.load_gather(wbuf, [jnp.zeros((16,), jnp.int32)])
    m1 = plsc.load_gather(wbuf, [jnp.full((16,), 16, jnp.int32)])
    for i in range(1, 16):
        m0 = jnp.maximum(m0, plsc.load_gather(
            wbuf, [jnp.full((16,), i, jnp.int32)]))
        m1 = jnp.maximum(m1, plsc.load_gather(
            wbuf, [jnp.full((16,), 16 + i, jnp.int32)]))
    shift = jnp.maximum(m0 + m1, jnp.zeros((16,), jnp.float32))

    zf = jnp.zeros((16,), jnp.float32)
    zi = jnp.zeros((16,), jnp.int32)
    for rr in range(CH):
        w32[rr, 0:16] = zf
        w32[rr, 16:32] = zf
    for g in range(CH // 16):
        cbuf[pl.ds(g * 16, 16)] = zi
    pltpu.sync_copy(w32.at[pl.ds(0, NDT), :],
                    den_sh.at[pl.ds(t * NDT, NDT), :])

    plsc.subcore_barrier()

    big = jnp.int32(1 << 30)
    ebase = wid * EPW
    SUPW = 512

    def _esuper(k, _):
        offL = ebase + k * SUPW
        pltpu.sync_copy(src.at[pl.ds(offL, SUPW)], sbufL)
        pltpu.sync_copy(dst.at[pl.ds(offL, SUPW)], dbufL)
        for j in range(SUPW // CH):
            off = j * CH
            vfrom = jnp.minimum(jnp.maximum(E - (offL + off), 0), big)
            _w_groups(as_t, ad_t, sbufL, dbufL, wbufL, w32, cbuf, d2buf,
                      shift, vfrom, off)
            pltpu.sync_copy(w32, den_sh.at[d2buf], add=True)
        pltpu.sync_copy(wbufL, w_all.at[pl.ds(offL, SUPW)])
        return 0

    lax.fori_loop(0, EPW // SUPW, _esuper, 0)

    sbase = wid * SELF_STRIDE

    def _schunk(k, _):
        j0 = k * CH
        for g in range(CH // 16):
            row0 = g * 16
            ids = sbase + j0 + row0 + _iota16()
            ids = jnp.where(ids < N, ids, 0)
            sbuf[pl.ds(row0, 16)] = ids
            dbuf[pl.ds(row0, 16)] = ids
        vfrom = jnp.minimum(SELF_STRIDE, N - sbase) - j0
        _w_groups(as_t, ad_t, sbuf, dbuf, wbuf, w32, cbuf, d2buf, shift,
                  vfrom, 0)
        pltpu.sync_copy(wbuf, w_all.at[pl.ds(E_PAD + sbase + j0, CH)])
        pltpu.sync_copy(w32, den_sh.at[d2buf], add=True)
        return 0

    lax.fori_loop(0, SELF_STRIDE // CH, _schunk, 0)

    plsc.subcore_barrier()
    pltpu.sync_copy(den_sh.at[pl.ds(t * NDT, NDT), :],
                    denf.at[pl.ds(c * ND + t * NDT, NDT), :])


@functools.partial(
    pl.kernel,
    out_type=[jax.ShapeDtypeStruct((W_LEN,), jnp.float32),
              jax.ShapeDtypeStruct((2 * ND, DW), jnp.float32)],
    mesh=_MESH,
    compiler_params=_SC_PARAMS,
    scratch_types=[
        pltpu.VMEM((N,), jnp.float32),
        pltpu.VMEM((N,), jnp.float32),
        pltpu.VMEM((512,), jnp.int32),
        pltpu.VMEM((512,), jnp.int32),
        pltpu.VMEM((512,), jnp.float32),
        pltpu.VMEM((CH,), jnp.int32),
        pltpu.VMEM((CH,), jnp.int32),
        pltpu.VMEM((CH,), jnp.float32),
        pltpu.VMEM((CH, DW), jnp.float32),
        pltpu.VMEM((CH,), jnp.int32),
        pltpu.VMEM((CH,), jnp.int32),
        pltpu.VMEM_SHARED((ND, DW), jnp.float32),
    ],
)
def _wden_kernel(asv, adv, src, dst, w_all, denf,
                 as_t, ad_t, sbufL, dbufL, wbufL, sbuf, dbuf, wbuf, w32,
                 cbuf, d2buf, den_sh):
    _wden_body(asv, adv, src, dst, w_all, denf,
               as_t, ad_t, sbufL, dbufL, wbufL, sbuf, dbuf, wbuf, w32,
               cbuf, d2buf, den_sh)


def _acc_body(hp4f, src, dst, w_all, accf,
              sbufL, dbufL, wbufL, sbuf, dbuf, wbuf, didx0, didx1,
              rows0, rows1, rowsS, acc_sh, gsem0, gsem1, ssem0, ssem1,
              sems):
    c = lax.axis_index("c")
    t = lax.axis_index("s")
    cN = c * NP
    r0 = t * NPT
    zf = jnp.zeros((16,), jnp.float32)
    ebase = t * EPG
    sbase = t * SPG
    SUP = 1024

    def _scale(rows, wsrc, woff):
        for g in range(CH // 16):
            wv = wsrc[pl.ds(woff + g * 16, 16)]
            for r in range(16):
                row = g * 16 + r
                wb = jnp.full((16,), wv[r], jnp.float32)
                rows[row, 0:16] = rows[row, 0:16] * wb

    for ps in (0, 1):
        pofs = 2 * ps * NP + cN

        for rr in range(CH):
            rowsS[rr, 0:16] = zf
        for j in range(NPT // CH):
            pltpu.sync_copy(rowsS, acc_sh.at[pl.ds(r0 + j * CH, CH), :])
        pltpu.sync_copy(rowsS.at[pl.ds(0, NPT % CH), :],
                        acc_sh.at[pl.ds(r0 + (NPT // CH) * CH, NPT % CH), :])

        plsc.subcore_barrier()

        def _esuper(k, _, pofs=pofs):
            offL = ebase + k * SUP
            pltpu.sync_copy(src.at[pl.ds(offL, SUP)], sbufL)
            pltpu.sync_copy(dst.at[pl.ds(offL, SUP)], dbufL)
            pltpu.sync_copy(w_all.at[pl.ds(offL, SUP)], wbufL)
            for g in range(SUP // 16):
                sbufL[pl.ds(g * 16, 16)] = sbufL[pl.ds(g * 16, 16)] + pofs

            def _pair_step(jj, _):
                a = jj * 2 * CH
                b = a + CH
                for g in range(CH // 16):
                    didx0[pl.ds(g * 16, 16)] = dbufL[pl.ds(a + g * 16, 16)]
                ga = pltpu.async_copy(hp4f.at[sbufL.at[pl.ds(a, CH)]],
                                      rows0, gsem0)
                for g in range(CH // 16):
                    didx1[pl.ds(g * 16, 16)] = dbufL[pl.ds(b + g * 16, 16)]
                gb = pltpu.async_copy(hp4f.at[sbufL.at[pl.ds(b, CH)]],
                                      rows1, gsem1)
                ga.wait()
                _scale(rows0, wbufL, a)
                sa = pltpu.async_copy(rows0, acc_sh.at[didx0], ssem0,
                                      add=True)
                gb.wait()
                _scale(rows1, wbufL, b)
                sb = pltpu.async_copy(rows1, acc_sh.at[didx1], ssem1,
                                      add=True)
                sa.wait()
                sb.wait()
                return 0

            lax.fori_loop(0, SUP // (2 * CH), _pair_step, 0)
            return 0

        lax.fori_loop(0, EPG // SUP, _esuper, 0)

        def _schunk(k, _, pofs=pofs):
            j0 = k * CH
            for g in range(CH // 16):
                row0 = g * 16
                ids = sbase + j0 + row0 + _iota16()
                ids = jnp.where(ids < N, ids, 0)
                sbuf[pl.ds(row0, 16)] = ids + pofs
                dbuf[pl.ds(row0, 16)] = ids
            pltpu.sync_copy(w_all.at[pl.ds(E_PAD + sbase + j0, CH)], wbuf)
            pltpu.async_copy(hp4f.at[sbuf], rowsS, sems).wait()
            _scale(rowsS, wbuf, 0)
            pltpu.sync_copy(rowsS, acc_sh.at[dbuf], add=True)
            return 0

        lax.fori_loop(0, SPG // CH, _schunk, 0)

        plsc.subcore_barrier()
        pltpu.sync_copy(acc_sh.at[pl.ds(r0, NPT), :],
                        accf.at[pl.ds(pofs + r0, NPT), :])


@functools.partial(
    pl.kernel,
    out_type=jax.ShapeDtypeStruct((4 * NP, HH), jnp.float32),
    mesh=_MESH,
    compiler_params=_SC_PARAMS,
    scratch_types=[
        pltpu.VMEM((1024,), jnp.int32),
        pltpu.VMEM((1024,), jnp.int32),
        pltpu.VMEM((1024,), jnp.float32),
        pltpu.VMEM((CH,), jnp.int32),
        pltpu.VMEM((CH,), jnp.int32),
        pltpu.VMEM((CH,), jnp.float32),
        pltpu.VMEM((CH,), jnp.int32),
        pltpu.VMEM((CH,), jnp.int32),
        pltpu.VMEM((CH, HH), jnp.float32),
        pltpu.VMEM((CH, HH), jnp.float32),
        pltpu.VMEM((CH, HH), jnp.float32),
        pltpu.VMEM_SHARED((NP, HH), jnp.float32),
        pltpu.SemaphoreType.DMA,
        pltpu.SemaphoreType.DMA,
        pltpu.SemaphoreType.DMA,
        pltpu.SemaphoreType.DMA,
        pltpu.SemaphoreType.DMA,
    ],
)
def _acc_kernel(hp4f, src, dst, w_all, accf,
                sbufL, dbufL, wbufL, sbuf, dbuf, wbuf, didx0, didx1,
                rows0, rows1, rowsS, acc_sh, gsem0, gsem1, ssem0, ssem1,
                sems):
    _acc_body(hp4f, src, dst, w_all, accf,
              sbufL, dbufL, wbufL, sbuf, dbuf, wbuf, didx0, didx1,
              rows0, rows1, rowsS, acc_sh, gsem0, gsem1, ssem0, ssem1,
              sems)


EPT2 = E // 32


def _pair_body(g1, g2, src, dst, zs, zd, sbufL, dbufL,
               r1a, r2a, r1b, r2b,
               sg1a, sg2a, sw1a, sw2a, sg1b, sg2b, sw1b, sw2b):
    c = lax.axis_index("c")
    t = lax.axis_index("s")
    wid = t * 2 + c
    base = wid * EPT2
    SUPP = 512
    nsup = EPT2 // SUPP

    def _gfire(off, r1, r2, s1, s2):
        d1 = pltpu.async_copy(g1.at[sbufL.at[pl.ds(off, CH)]], r1, s1)
        d2 = pltpu.async_copy(g2.at[dbufL.at[pl.ds(off, CH)]], r2, s2)
        return d1, d2

    def _wfire(off, r1, r2, s1, s2):
        d1 = pltpu.async_copy(r1, zs.at[pl.ds(off, CH), :], s1)
        d2 = pltpu.async_copy(r2, zd.at[pl.ds(off, CH), :], s2)
        return d1, d2

    def _super(k, _):
        offL = base + k * SUPP
        pltpu.sync_copy(src.at[pl.ds(offL, SUPP)], sbufL)
        pltpu.sync_copy(dst.at[pl.ds(offL, SUPP)], dbufL)
        g0 = _gfire(0, r1a, r2a, sg1a, sg2a)
        g1d = _gfire(CH, r1b, r2b, sg1b, sg2b)
        g0[0].wait(); g0[1].wait()
        w0 = _wfire(offL, r1a, r2a, sw1a, sw2a)
        g1d[0].wait(); g1d[1].wait()
        w1 = _wfire(offL + CH, r1b, r2b, sw1b, sw2b)
        w0[0].wait(); w0[1].wait()
        g2d = _gfire(2 * CH, r1a, r2a, sg1a, sg2a)
        g2d[0].wait(); g2d[1].wait()
        w2 = _wfire(offL + 2 * CH, r1a, r2a, sw1a, sw2a)
        w1[0].wait(); w1[1].wait()
        g3 = _gfire(3 * CH, r1b, r2b, sg1b, sg2b)
        g3[0].wait(); g3[1].wait()
        w3 = _wfire(offL + 3 * CH, r1b, r2b, sw1b, sw2b)
        w2[0].wait(); w2[1].wait()
        w3[0].wait(); w3[1].wait()
        return 0

    lax.fori_loop(0, nsup, _super, 0)

    def _tchunk(off, nreal):
        if nreal < CH:
            zi = jnp.zeros((16,), jnp.int32)
            for g in range(CH // 16):
                sbufL[pl.ds(g * 16, 16)] = zi
                dbufL[pl.ds(g * 16, 16)] = zi
        pltpu.sync_copy(src.at[pl.ds(off, nreal)], sbufL.at[pl.ds(0, nreal)])
        pltpu.sync_copy(dst.at[pl.ds(off, nreal)], dbufL.at[pl.ds(0, nreal)])
        pltpu.async_copy(g1.at[sbufL.at[pl.ds(0, CH)]], r1a, sg1a).wait()
        pltpu.async_copy(g2.at[dbufL.at[pl.ds(0, CH)]], r2a, sg2a).wait()
        pltpu.sync_copy(r1a.at[pl.ds(0, nreal), :], zs.at[pl.ds(off, nreal), :])
        pltpu.sync_copy(r2a.at[pl.ds(0, nreal), :], zd.at[pl.ds(off, nreal), :])

    tbase = base + nsup * SUPP
    _tchunk(tbase, CH)
    _tchunk(tbase + CH, CH)
    _tchunk(tbase + 2 * CH, CH)
    _tchunk(tbase + 3 * CH, 40)


@functools.partial(
    pl.kernel,
    out_type=[jax.ShapeDtypeStruct((E, H), jnp.float32),
              jax.ShapeDtypeStruct((E, H), jnp.float32)],
    mesh=_MESH,
    compiler_params=_SC_PARAMS,
    scratch_types=[
        pltpu.VMEM((512,), jnp.int32),
        pltpu.VMEM((512,), jnp.int32),
        pltpu.VMEM((CH, H), jnp.float32),
        pltpu.VMEM((CH, H), jnp.float32),
        pltpu.VMEM((CH, H), jnp.float32),
        pltpu.VMEM((CH, H), jnp.float32),
        pltpu.SemaphoreType.DMA,
        pltpu.SemaphoreType.DMA,
        pltpu.SemaphoreType.DMA,
        pltpu.SemaphoreType.DMA,
        pltpu.SemaphoreType.DMA,
        pltpu.SemaphoreType.DMA,
        pltpu.SemaphoreType.DMA,
        pltpu.SemaphoreType.DMA,
    ],
)
def _pair_kernel(g1, g2, src, dst, zs, zd, sbufL, dbufL,
                 r1a, r2a, r1b, r2b,
                 sg1a, sg2a, sw1a, sw2a, sg1b, sg2b, sw1b, sw2b):
    _pair_body(g1, g2, src, dst, zs, zd, sbufL, dbufL,
               r1a, r2a, r1b, r2b,
               sg1a, sg2a, sw1a, sw2a, sg1b, sg2b, sw1b, sw2b)


def kernel(x, edge_index, W0, a_s0, a_d0, b0, Ws, a_ss, a_ds, bs,
           pp_W1, pp_b1, pp_W2, pp_b2, pp_W3, pp_b3,
           vp_W1, vp_b1, vp_W2, vp_b2, lp_W1, lp_b1, lp_W2, lp_b2):
    src = edge_index[0]
    dst = edge_index[1]
    pad = jnp.zeros((E_PAD - E,), jnp.int32)
    src_p = jnp.concatenate([src, pad])
    dst_p = jnp.concatenate([dst, pad])
    x_p = jnp.concatenate([x, jnp.zeros((NP - N, x.shape[1]), x.dtype)])

    hp4, sv, dv = _enc0_call(x_p, W0, a_s0.reshape(H, 1), a_d0.reshape(H, 1))

    w_xs = jnp.stack([Ws[0], Ws[1], Ws[2], Ws[2]])
    as_xs = jnp.concatenate([a_ss, a_ss[2:3]]).reshape(4, H, 1)
    ad_xs = jnp.concatenate([a_ds, a_ds[2:3]]).reshape(4, H, 1)
    b_xs = jnp.stack([b0, bs[0], bs[1], bs[2]]).reshape(4, 1, H)

    def _layer_step(carry, xs):
        hp4, sv, dv = carry[:3]
        w_all, denf = _wden_kernel(sv, dv, src_p, dst_p)
        accf = _acc_kernel(hp4.reshape(4 * NP, HH), src_p, dst_p, w_all)
        quarters = accf.reshape(4, NP, HH)
        den_a = denf[:ND].reshape(ND * DW, 1)[:NP]
        den_b = denf[ND:].reshape(ND * DW, 1)[:NP]
        hp4n, svn, dvn = _enc_call(
            quarters[0], quarters[1], quarters[2], quarters[3], den_a, den_b,
            xs["b"], xs["W"], xs["a_s"], xs["a_d"])
        return ((hp4n, svn.reshape(NP), dvn.reshape(NP), quarters,
                 den_a, den_b), None)

    init = (hp4, sv.reshape(NP), dv.reshape(NP),
            jnp.zeros((4, NP, HH), jnp.float32),
            jnp.zeros((NP, 1), jnp.float32),
            jnp.zeros((NP, 1), jnp.float32))
    carry, _ = lax.scan(
        _layer_step, init,
        {"W": w_xs, "a_s": as_xs, "a_d": ad_xs, "b": b_xs})
    quarters = carry[3]
    den_a, den_b = carry[4], carry[5]

    nf, g1, g2, vias, layers = _heads_call(
        quarters[0], quarters[1], quarters[2], quarters[3], den_a, den_b,
        bs[2].reshape(1, H),
        pp_W1[:H, :], pp_W1[H:, :], pp_b1.reshape(1, H),
        vp_W1, vp_b1.reshape(1, 32), vp_W2, vp_b2.reshape(1, 1),
        lp_W1, lp_b1.reshape(1, 32), lp_W2, lp_b2.reshape(1, 4))

    zs, zd = _pair_kernel(g1, g2, src, dst)
    paths = _path_call(zs, zd, pp_W2, pp_b2.reshape(1, 32),
                       pp_W3, pp_b3.reshape(1, 1))

    return (paths.reshape(E), vias.reshape(NP)[:N], layers[:N], nf[:N])
